# Initial kernel scaffold; baseline (speedup 1.0000x reference)
import jax
import jax.numpy as jnp
from jax.experimental import pallas as pl


def kernel(x, edge_index, batch, W1, b1, W2, b2, W3, b3, Wl, bl):
    raise NotImplementedError("write your pallas kernel here")


# speedup vs baseline: 12.3886x; 12.3886x over previous
import functools

import jax
import jax.numpy as jnp
from jax import lax
from jax.experimental import pallas as pl
from jax.experimental.pallas import tpu as pltpu
from jax.experimental.pallas import tpu_sc as plsc

N = 10000
E = 320000
N_X = 128
DIM_H = 64
N_Y = 10

NC, NS = 2, 16
NW = NC * NS
CH = 128
CHUNKS = 80
E_PAD = NW * CHUNKS * CH
N_PAD = 10240
ROWS_PER_TILE = N_PAD // NS

BLK = 1024
GRID = N_PAD // BLK


def _mesh():
    return plsc.VectorSubcoreMesh(core_axis_name="c", subcore_axis_name="s")


def _sc_deg(dst_r, emask):
    @functools.partial(
        pl.kernel,
        out_type=jax.ShapeDtypeStruct((NW, N_PAD), jnp.float32),
        mesh=_mesh(),
        compiler_params=pltpu.CompilerParams(needs_layout_passes=False),
        scratch_types=[
            pltpu.VMEM((CHUNKS, CH), jnp.int32),
            pltpu.VMEM((CHUNKS, CH), jnp.float32),
            pltpu.VMEM((N_PAD,), jnp.float32),
        ],
    )
    def body(dst_hbm, mask_hbm, out_hbm, idx_d, mvals, deg):
        c = lax.axis_index("c")
        s = lax.axis_index("s")
        wid = c * NS + s
        pltpu.sync_copy(dst_hbm.at[wid], idx_d)
        pltpu.sync_copy(mask_hbm.at[wid], mvals)
        zero16 = jnp.zeros((16,), jnp.float32)

        def zbody(i, carry):
            deg[pl.ds(i * 16, 16)] = zero16
            return carry

        lax.fori_loop(0, N_PAD // 16, zbody, 0)

        def cbody(j, carry):
            for g in range(CH // 16):
                idx16 = idx_d[j, pl.ds(g * 16, 16)]
                m16 = mvals[j, pl.ds(g * 16, 16)]
                plsc.addupdate_scatter(deg, [idx16], m16)
            return carry

        lax.fori_loop(0, CHUNKS, cbody, 0)
        pltpu.sync_copy(deg, out_hbm.at[wid])

    return body(dst_r, emask)


def _sc_agg(y, src_r, dst_r):
    @functools.partial(
        pl.kernel,
        out_type=jax.ShapeDtypeStruct((NC, N_PAD, DIM_H), jnp.float32),
        mesh=_mesh(),
        compiler_params=pltpu.CompilerParams(
            needs_layout_passes=False, use_tc_tiling_on_sc=False
        ),
        scratch_types=[
            pltpu.VMEM((CHUNKS, CH), jnp.int32),
            pltpu.VMEM((CHUNKS, CH), jnp.int32),
            pltpu.VMEM((CH, DIM_H), jnp.float32),
            pltpu.VMEM_SHARED((N_PAD, DIM_H), jnp.float32),
            pltpu.SemaphoreType.DMA,
        ],
    )
    def body(y_hbm, src_hbm, dst_hbm, out_hbm, idx_s, idx_d, rows, acc, sem):
        c = lax.axis_index("c")
        s = lax.axis_index("s")
        wid = c * NS + s
        pltpu.sync_copy(src_hbm.at[wid], idx_s)
        pltpu.sync_copy(dst_hbm.at[wid], idx_d)

        zero16 = jnp.zeros((16,), jnp.float32)

        def zbody(i, carry):
            for k in range(DIM_H // 16):
                rows[i, pl.ds(k * 16, 16)] = zero16
            return carry

        lax.fori_loop(0, CH, zbody, 0)
        base = s * ROWS_PER_TILE
        for k in range(ROWS_PER_TILE // CH):
            pltpu.sync_copy(rows, acc.at[pl.ds(base + k * CH, CH)])
        plsc.subcore_barrier()

        def cbody(j, carry):
            pltpu.async_copy(y_hbm.at[idx_s.at[j]], rows, sem).wait()
            pltpu.sync_copy(rows, acc.at[idx_d.at[j]], add=True)
            return carry

        lax.fori_loop(0, CHUNKS, cbody, 0)
        plsc.subcore_barrier()
        for k in range(ROWS_PER_TILE // CH):
            pltpu.sync_copy(acc.at[pl.ds(base + k * CH, CH)], rows)
            pltpu.sync_copy(rows, out_hbm.at[c, pl.ds(base + k * CH, CH)])

    return body(y, src_r, dst_r)


def _tc_head(deg_p, x_pad, W1):
    def body(deg_ref, x_ref, w_ref, y_ref, dinv_ref):
        i = pl.program_id(0)
        degt = jnp.transpose(deg_ref[...])
        deg = jnp.sum(degt, axis=1, keepdims=True)
        rowid = i * BLK + lax.broadcasted_iota(jnp.int32, (BLK, 1), 0)
        deg = deg + (rowid < N).astype(jnp.float32)
        dinv = jnp.where(deg > 0, lax.rsqrt(deg), 0.0)
        xw = jnp.dot(x_ref[...], w_ref[...], preferred_element_type=jnp.float32)
        y_ref[...] = xw * dinv
        dinv_ref[...] = dinv

    return pl.pallas_call(
        body,
        grid=(GRID,),
        in_specs=[
            pl.BlockSpec((NW, BLK), lambda i: (0, i)),
            pl.BlockSpec((BLK, N_X), lambda i: (i, 0)),
            pl.BlockSpec((N_X, DIM_H), lambda i: (0, 0)),
        ],
        out_specs=[
            pl.BlockSpec((BLK, DIM_H), lambda i: (i, 0)),
            pl.BlockSpec((BLK, 1), lambda i: (i, 0)),
        ],
        out_shape=[
            jax.ShapeDtypeStruct((N_PAD, DIM_H), jnp.float32),
            jax.ShapeDtypeStruct((N_PAD, 1), jnp.float32),
        ],
    )(deg_p, x_pad, W1)


def _tc_mid(p, y, dinv, b, W):
    def body(p_ref, y_ref, dinv_ref, b_ref, w_ref, o_ref):
        agg = jnp.sum(p_ref[...], axis=0)
        dinv = dinv_ref[...]
        h = jnp.maximum(dinv * (agg + y_ref[...]) + b_ref[...], 0.0)
        o_ref[...] = jnp.dot(h, w_ref[...], preferred_element_type=jnp.float32) * dinv

    return pl.pallas_call(
        body,
        grid=(GRID,),
        in_specs=[
            pl.BlockSpec((NC, BLK, DIM_H), lambda i: (0, i, 0)),
            pl.BlockSpec((BLK, DIM_H), lambda i: (i, 0)),
            pl.BlockSpec((BLK, 1), lambda i: (i, 0)),
            pl.BlockSpec((1, DIM_H), lambda i: (0, 0)),
            pl.BlockSpec((DIM_H, DIM_H), lambda i: (0, 0)),
        ],
        out_specs=pl.BlockSpec((BLK, DIM_H), lambda i: (i, 0)),
        out_shape=jax.ShapeDtypeStruct((N_PAD, DIM_H), jnp.float32),
    )(p, y, dinv, b, W)


def _tc_tail(p, y, dinv, b, Wl, bl):
    def body(p_ref, y_ref, dinv_ref, b_ref, wl_ref, bl_ref, o_ref):
        agg = jnp.sum(p_ref[...], axis=0)
        dinv = dinv_ref[...]
        h = jnp.maximum(dinv * (agg + y_ref[...]) + b_ref[...], 0.0)
        o_ref[...] = jnp.dot(h, wl_ref[...], preferred_element_type=jnp.float32) + bl_ref[...]

    return pl.pallas_call(
        body,
        grid=(GRID,),
        in_specs=[
            pl.BlockSpec((NC, BLK, DIM_H), lambda i: (0, i, 0)),
            pl.BlockSpec((BLK, DIM_H), lambda i: (i, 0)),
            pl.BlockSpec((BLK, 1), lambda i: (i, 0)),
            pl.BlockSpec((1, DIM_H), lambda i: (0, 0)),
            pl.BlockSpec((DIM_H, N_Y), lambda i: (0, 0)),
            pl.BlockSpec((1, N_Y), lambda i: (0, 0)),
        ],
        out_specs=pl.BlockSpec((BLK, N_Y), lambda i: (i, 0)),
        out_shape=jax.ShapeDtypeStruct((N_PAD, N_Y), jnp.float32),
    )(p, y, dinv, b, Wl, bl)


def kernel(x, edge_index, batch, W1, b1, W2, b2, W3, b3, Wl, bl):
    del batch
    src = edge_index[0].astype(jnp.int32)
    dst = edge_index[1].astype(jnp.int32)
    pad = E_PAD - E
    src_r = jnp.concatenate([src, jnp.full((pad,), N, jnp.int32)]).reshape(NW, CHUNKS, CH)
    dst_r = jnp.concatenate([dst, jnp.zeros((pad,), jnp.int32)]).reshape(NW, CHUNKS, CH)
    emask = jnp.concatenate(
        [jnp.ones((E,), jnp.float32), jnp.zeros((pad,), jnp.float32)]
    ).reshape(NW, CHUNKS, CH)
    x_pad = jnp.pad(x, ((0, N_PAD - N), (0, 0)))

    deg_p = _sc_deg(dst_r, emask)
    y1, dinv = _tc_head(deg_p, x_pad, W1)
    p1 = _sc_agg(y1, src_r, dst_r)
    y2 = _tc_mid(p1, y1, dinv, b1.reshape(1, -1), W2)
    p2 = _sc_agg(y2, src_r, dst_r)
    y3 = _tc_mid(p2, y2, dinv, b2.reshape(1, -1), W3)
    p3 = _sc_agg(y3, src_r, dst_r)
    out = _tc_tail(p3, y3, dinv, b3.reshape(1, -1), Wl, bl.reshape(1, -1))
    return out[:N]


# speedup vs baseline: 14.7555x; 1.1911x over previous
import functools

import jax
import jax.numpy as jnp
from jax import lax
from jax.experimental import pallas as pl
from jax.experimental.pallas import tpu as pltpu
from jax.experimental.pallas import tpu_sc as plsc

N = 10000
E = 320000
N_X = 128
DIM_H = 64
N_Y = 10

NC, NS = 2, 16
NW = NC * NS
CH = 128
CHUNKS = 80
E_PAD = NW * CHUNKS * CH
N_PAD = 10240
ROWS_PER_TILE = N_PAD // NS

BLK = 1024
GRID = N_PAD // BLK

NBUF = 8
K_AHEAD = 4


def _mesh():
    return plsc.VectorSubcoreMesh(core_axis_name="c", subcore_axis_name="s")


def _sc_deg(dst_r, emask):
    @functools.partial(
        pl.kernel,
        out_type=jax.ShapeDtypeStruct((NW, N_PAD), jnp.float32),
        mesh=_mesh(),
        compiler_params=pltpu.CompilerParams(needs_layout_passes=False),
        scratch_types=[
            pltpu.VMEM((CHUNKS, CH), jnp.int32),
            pltpu.VMEM((CHUNKS, CH), jnp.float32),
            pltpu.VMEM((N_PAD,), jnp.float32),
        ],
    )
    def body(dst_hbm, mask_hbm, out_hbm, idx_d, mvals, deg):
        c = lax.axis_index("c")
        s = lax.axis_index("s")
        wid = c * NS + s
        pltpu.sync_copy(dst_hbm.at[wid], idx_d)
        pltpu.sync_copy(mask_hbm.at[wid], mvals)
        zero16 = jnp.zeros((16,), jnp.float32)

        def zbody(i, carry):
            deg[pl.ds(i * 16, 16)] = zero16
            return carry

        lax.fori_loop(0, N_PAD // 16, zbody, 0)

        def cbody(j, carry):
            for g in range(CH // 16):
                idx16 = idx_d[j, pl.ds(g * 16, 16)]
                m16 = mvals[j, pl.ds(g * 16, 16)]
                plsc.addupdate_scatter(deg, [idx16], m16)
            return carry

        lax.fori_loop(0, CHUNKS, cbody, 0)
        pltpu.sync_copy(deg, out_hbm.at[wid])

    return body(dst_r, emask)


def _sc_agg(y, src_r, dst_r):
    @functools.partial(
        pl.kernel,
        out_type=jax.ShapeDtypeStruct((NC, N_PAD, DIM_H), jnp.float32),
        mesh=_mesh(),
        compiler_params=pltpu.CompilerParams(
            needs_layout_passes=False, use_tc_tiling_on_sc=False
        ),
        scratch_types=[
            pltpu.VMEM((CHUNKS, CH), jnp.int32),
            pltpu.VMEM((CHUNKS, CH), jnp.int32),
            pltpu.VMEM((NBUF, CH, DIM_H), jnp.float32),
            pltpu.VMEM_SHARED((N_PAD, DIM_H), jnp.float32),
            pltpu.SemaphoreType.DMA((NBUF,)),
            pltpu.SemaphoreType.DMA((NBUF,)),
        ],
    )
    def body(y_hbm, src_hbm, dst_hbm, out_hbm, idx_s, idx_d, rows, acc, sem_g, sem_s):
        c = lax.axis_index("c")
        s = lax.axis_index("s")
        wid = c * NS + s
        pltpu.sync_copy(src_hbm.at[wid], idx_s)
        pltpu.sync_copy(dst_hbm.at[wid], idx_d)

        zero16 = jnp.zeros((16,), jnp.float32)

        def zbody(i, carry):
            for k in range(DIM_H // 16):
                rows[0, i, pl.ds(k * 16, 16)] = zero16
            return carry

        lax.fori_loop(0, CH, zbody, 0)
        base = s * ROWS_PER_TILE
        for k in range(ROWS_PER_TILE // CH):
            pltpu.sync_copy(rows.at[0], acc.at[pl.ds(base + k * CH, CH)])
        plsc.subcore_barrier()

        for j in range(K_AHEAD):
            pltpu.async_copy(y_hbm.at[idx_s.at[j]], rows.at[j], sem_g.at[j])

        def _wait(sem, b):
            pltpu.make_async_copy(y_hbm.at[pl.ds(0, CH)], rows.at[b], sem).wait()

        def cbody(j, carry):
            b = lax.rem(j, NBUF)
            _wait(sem_g.at[b], b)
            pltpu.async_copy(rows.at[b], acc.at[idx_d.at[j]], sem_s.at[b], add=True)
            jn = j + K_AHEAD
            bn = lax.rem(jn, NBUF)

            @pl.when(jn < CHUNKS)
            def _():
                @pl.when(jn >= NBUF)
                def _():
                    _wait(sem_s.at[bn], bn)
                pltpu.async_copy(y_hbm.at[idx_s.at[jn]], rows.at[bn], sem_g.at[bn])

            return carry

        lax.fori_loop(0, CHUNKS, cbody, 0)
        for b in range(NBUF):
            _wait(sem_s.at[b], b)
        plsc.subcore_barrier()
        for k in range(ROWS_PER_TILE // CH):
            pltpu.sync_copy(acc.at[pl.ds(base + k * CH, CH)], rows.at[0])
            pltpu.sync_copy(rows.at[0], out_hbm.at[c, pl.ds(base + k * CH, CH)])

    return body(y, src_r, dst_r)


def _tc_head(deg_p, x_pad, W1):
    def body(deg_ref, x_ref, w_ref, y_ref, dinv_ref):
        i = pl.program_id(0)
        degt = jnp.transpose(deg_ref[...])
        deg = jnp.sum(degt, axis=1, keepdims=True)
        rowid = i * BLK + lax.broadcasted_iota(jnp.int32, (BLK, 1), 0)
        deg = deg + (rowid < N).astype(jnp.float32)
        dinv = jnp.where(deg > 0, lax.rsqrt(deg), 0.0)
        xw = jnp.dot(x_ref[...], w_ref[...], preferred_element_type=jnp.float32)
        y_ref[...] = xw * dinv
        dinv_ref[...] = dinv

    return pl.pallas_call(
        body,
        grid=(GRID,),
        in_specs=[
            pl.BlockSpec((NW, BLK), lambda i: (0, i)),
            pl.BlockSpec((BLK, N_X), lambda i: (i, 0)),
            pl.BlockSpec((N_X, DIM_H), lambda i: (0, 0)),
        ],
        out_specs=[
            pl.BlockSpec((BLK, DIM_H), lambda i: (i, 0)),
            pl.BlockSpec((BLK, 1), lambda i: (i, 0)),
        ],
        out_shape=[
            jax.ShapeDtypeStruct((N_PAD, DIM_H), jnp.float32),
            jax.ShapeDtypeStruct((N_PAD, 1), jnp.float32),
        ],
    )(deg_p, x_pad, W1)


def _tc_mid(p, y, dinv, b, W):
    def body(p_ref, y_ref, dinv_ref, b_ref, w_ref, o_ref):
        agg = jnp.sum(p_ref[...], axis=0)
        dinv = dinv_ref[...]
        h = jnp.maximum(dinv * (agg + y_ref[...]) + b_ref[...], 0.0)
        o_ref[...] = jnp.dot(h, w_ref[...], preferred_element_type=jnp.float32) * dinv

    return pl.pallas_call(
        body,
        grid=(GRID,),
        in_specs=[
            pl.BlockSpec((NC, BLK, DIM_H), lambda i: (0, i, 0)),
            pl.BlockSpec((BLK, DIM_H), lambda i: (i, 0)),
            pl.BlockSpec((BLK, 1), lambda i: (i, 0)),
            pl.BlockSpec((1, DIM_H), lambda i: (0, 0)),
            pl.BlockSpec((DIM_H, DIM_H), lambda i: (0, 0)),
        ],
        out_specs=pl.BlockSpec((BLK, DIM_H), lambda i: (i, 0)),
        out_shape=jax.ShapeDtypeStruct((N_PAD, DIM_H), jnp.float32),
    )(p, y, dinv, b, W)


def _tc_tail(p, y, dinv, b, Wl, bl):
    def body(p_ref, y_ref, dinv_ref, b_ref, wl_ref, bl_ref, o_ref):
        agg = jnp.sum(p_ref[...], axis=0)
        dinv = dinv_ref[...]
        h = jnp.maximum(dinv * (agg + y_ref[...]) + b_ref[...], 0.0)
        o_ref[...] = jnp.dot(h, wl_ref[...], preferred_element_type=jnp.float32) + bl_ref[...]

    return pl.pallas_call(
        body,
        grid=(GRID,),
        in_specs=[
            pl.BlockSpec((NC, BLK, DIM_H), lambda i: (0, i, 0)),
            pl.BlockSpec((BLK, DIM_H), lambda i: (i, 0)),
            pl.BlockSpec((BLK, 1), lambda i: (i, 0)),
            pl.BlockSpec((1, DIM_H), lambda i: (0, 0)),
            pl.BlockSpec((DIM_H, N_Y), lambda i: (0, 0)),
            pl.BlockSpec((1, N_Y), lambda i: (0, 0)),
        ],
        out_specs=pl.BlockSpec((BLK, N_Y), lambda i: (i, 0)),
        out_shape=jax.ShapeDtypeStruct((N_PAD, N_Y), jnp.float32),
    )(p, y, dinv, b, Wl, bl)


def kernel(x, edge_index, batch, W1, b1, W2, b2, W3, b3, Wl, bl):
    del batch
    src = edge_index[0].astype(jnp.int32)
    dst = edge_index[1].astype(jnp.int32)
    pad = E_PAD - E
    src_r = jnp.concatenate([src, jnp.full((pad,), N, jnp.int32)]).reshape(NW, CHUNKS, CH)
    dst_r = jnp.concatenate([dst, jnp.zeros((pad,), jnp.int32)]).reshape(NW, CHUNKS, CH)
    emask = jnp.concatenate(
        [jnp.ones((E,), jnp.float32), jnp.zeros((pad,), jnp.float32)]
    ).reshape(NW, CHUNKS, CH)
    x_pad = jnp.pad(x, ((0, N_PAD - N), (0, 0)))

    deg_p = _sc_deg(dst_r, emask)
    y1, dinv = _tc_head(deg_p, x_pad, W1)
    p1 = _sc_agg(y1, src_r, dst_r)
    y2 = _tc_mid(p1, y1, dinv, b1.reshape(1, -1), W2)
    p2 = _sc_agg(y2, src_r, dst_r)
    y3 = _tc_mid(p2, y2, dinv, b2.reshape(1, -1), W3)
    p3 = _sc_agg(y3, src_r, dst_r)
    out = _tc_tail(p3, y3, dinv, b3.reshape(1, -1), Wl, bl.reshape(1, -1))
    return out[:N]


# speedup vs baseline: 32.1220x; 2.1769x over previous
import functools

import jax
import jax.numpy as jnp
from jax import lax
from jax.experimental import pallas as pl
from jax.experimental.pallas import tpu as pltpu
from jax.experimental.pallas import tpu_sc as plsc

N = 10000
E = 320000
N_X = 128
DIM_H = 64
N_Y = 10

NC, NS = 2, 16
NW = NC * NS
HALF = DIM_H // NC
CH = 128
CHUNKS_T = 160
E_PAD = NS * CHUNKS_T * CH
N_PAD = 10240
ROWS_PER_TILE = N_PAD // NS

BLK = 1024
GRID = N_PAD // BLK

NBUF = 8
K_AHEAD = 4


def _mesh():
    return plsc.VectorSubcoreMesh(core_axis_name="c", subcore_axis_name="s")


_SC_PARAMS = dict(
    compiler_params=pltpu.CompilerParams(
        needs_layout_passes=False, use_tc_tiling_on_sc=False
    ),
)


def _sc_deg(dst_r, emask):
    @functools.partial(
        pl.kernel,
        out_type=jax.ShapeDtypeStruct((NW, N_PAD), jnp.float32),
        mesh=_mesh(),
        scratch_types=[
            pltpu.VMEM((CHUNKS_T // NC, CH), jnp.int32),
            pltpu.VMEM((CHUNKS_T // NC, CH), jnp.float32),
            pltpu.VMEM((N_PAD,), jnp.float32),
        ],
        **_SC_PARAMS,
    )
    def body(dst_hbm, mask_hbm, out_hbm, idx_d, mvals, deg):
        c = lax.axis_index("c")
        s = lax.axis_index("s")
        wid = c * NS + s
        half = CHUNKS_T // NC
        pltpu.sync_copy(dst_hbm.at[s, pl.ds(c * half, half)], idx_d)
        pltpu.sync_copy(mask_hbm.at[s, pl.ds(c * half, half)], mvals)
        zero16 = jnp.zeros((16,), jnp.float32)

        def zbody(i, carry):
            deg[pl.ds(i * 16, 16)] = zero16
            return carry

        lax.fori_loop(0, N_PAD // 16, zbody, 0)

        def cbody(j, carry):
            for g in range(CH // 16):
                idx16 = idx_d[j, pl.ds(g * 16, 16)]
                m16 = mvals[j, pl.ds(g * 16, 16)]
                plsc.addupdate_scatter(deg, [idx16], m16)
            return carry

        lax.fori_loop(0, half, cbody, 0)
        pltpu.sync_copy(deg, out_hbm.at[wid])

    return body(dst_r, emask)


def _sc_agg(y, src_r, dst_r):
    @functools.partial(
        pl.kernel,
        out_type=jax.ShapeDtypeStruct((NC, N_PAD, HALF), jnp.float32),
        mesh=_mesh(),
        scratch_types=[
            pltpu.VMEM((CHUNKS_T, CH), jnp.int32),
            pltpu.VMEM((CHUNKS_T, CH), jnp.int32),
            pltpu.VMEM((NBUF, CH, HALF), jnp.float32),
            pltpu.VMEM_SHARED((N_PAD, HALF), jnp.float32),
            pltpu.VMEM_SHARED((N_PAD, HALF), jnp.float32),
            pltpu.SemaphoreType.DMA((NBUF,)),
            pltpu.SemaphoreType.DMA((NBUF,)),
        ],
        **_SC_PARAMS,
    )
    def body(y_hbm, src_hbm, dst_hbm, out_hbm, idx_s, idx_d, rows, acc, y_sp, sem_g, sem_s):
        c = lax.axis_index("c")
        s = lax.axis_index("s")
        pltpu.sync_copy(src_hbm.at[s], idx_s)
        pltpu.sync_copy(dst_hbm.at[s], idx_d)

        zero16 = jnp.zeros((16,), jnp.float32)

        def zbody(i, carry):
            for k in range(HALF // 16):
                rows[0, i, pl.ds(k * 16, 16)] = zero16
            return carry

        lax.fori_loop(0, CH, zbody, 0)
        base = s * ROWS_PER_TILE
        for k in range(ROWS_PER_TILE // CH):
            pltpu.sync_copy(rows.at[0], acc.at[pl.ds(base + k * CH, CH)])
        for k in range(ROWS_PER_TILE // CH):
            pltpu.sync_copy(y_hbm.at[c, pl.ds(base + k * CH, CH)], rows.at[1])
            pltpu.sync_copy(rows.at[1], y_sp.at[pl.ds(base + k * CH, CH)])
        plsc.subcore_barrier()

        for j in range(K_AHEAD):
            pltpu.async_copy(y_sp.at[idx_s.at[j]], rows.at[j], sem_g.at[j])

        def _wait(sem, b):
            pltpu.make_async_copy(y_hbm.at[0, pl.ds(0, CH)], rows.at[b], sem).wait()

        def cbody(j, carry):
            b = lax.rem(j, NBUF)
            _wait(sem_g.at[b], b)
            pltpu.async_copy(rows.at[b], acc.at[idx_d.at[j]], sem_s.at[b], add=True)
            jn = j + K_AHEAD
            bn = lax.rem(jn, NBUF)

            @pl.when(jn < CHUNKS_T)
            def _():
                @pl.when(jn >= NBUF)
                def _():
                    _wait(sem_s.at[bn], bn)
                pltpu.async_copy(y_sp.at[idx_s.at[jn]], rows.at[bn], sem_g.at[bn])

            return carry

        lax.fori_loop(0, CHUNKS_T, cbody, 0)
        for b in range(NBUF):
            _wait(sem_s.at[b], b)
        plsc.subcore_barrier()
        for k in range(ROWS_PER_TILE // CH):
            pltpu.sync_copy(acc.at[pl.ds(base + k * CH, CH)], rows.at[0])
            pltpu.sync_copy(rows.at[0], out_hbm.at[c, pl.ds(base + k * CH, CH)])

    return body(y, src_r, dst_r)


def _split(v):
    return v[:, :HALF], v[:, HALF:]


def _tc_head(deg_p, x_pad, W1):
    def body(deg_ref, x_ref, w_ref, y_ref, dinv_ref):
        i = pl.program_id(0)
        degt = jnp.transpose(deg_ref[...])
        deg = jnp.sum(degt, axis=1, keepdims=True)
        rowid = i * BLK + lax.broadcasted_iota(jnp.int32, (BLK, 1), 0)
        deg = deg + (rowid < N).astype(jnp.float32)
        dinv = jnp.where(deg > 0, lax.rsqrt(deg), 0.0)
        xw = jnp.dot(x_ref[...], w_ref[...], preferred_element_type=jnp.float32)
        yl, yr = _split(xw * dinv)
        y_ref[0] = yl
        y_ref[1] = yr
        dinv_ref[...] = dinv

    return pl.pallas_call(
        body,
        grid=(GRID,),
        in_specs=[
            pl.BlockSpec((NW, BLK), lambda i: (0, i)),
            pl.BlockSpec((BLK, N_X), lambda i: (i, 0)),
            pl.BlockSpec((N_X, DIM_H), lambda i: (0, 0)),
        ],
        out_specs=[
            pl.BlockSpec((NC, BLK, HALF), lambda i: (0, i, 0)),
            pl.BlockSpec((BLK, 1), lambda i: (i, 0)),
        ],
        out_shape=[
            jax.ShapeDtypeStruct((NC, N_PAD, HALF), jnp.float32),
            jax.ShapeDtypeStruct((N_PAD, 1), jnp.float32),
        ],
    )(deg_p, x_pad, W1)


def _tc_mid(p, y, dinv, b, W):
    def body(p_ref, y_ref, dinv_ref, b_ref, w_ref, o_ref):
        agg = jnp.concatenate([p_ref[0], p_ref[1]], axis=-1)
        yv = jnp.concatenate([y_ref[0], y_ref[1]], axis=-1)
        dinv = dinv_ref[...]
        h = jnp.maximum(dinv * (agg + yv) + b_ref[...], 0.0)
        yl, yr = _split(jnp.dot(h, w_ref[...], preferred_element_type=jnp.float32) * dinv)
        o_ref[0] = yl
        o_ref[1] = yr

    return pl.pallas_call(
        body,
        grid=(GRID,),
        in_specs=[
            pl.BlockSpec((NC, BLK, HALF), lambda i: (0, i, 0)),
            pl.BlockSpec((NC, BLK, HALF), lambda i: (0, i, 0)),
            pl.BlockSpec((BLK, 1), lambda i: (i, 0)),
            pl.BlockSpec((1, DIM_H), lambda i: (0, 0)),
            pl.BlockSpec((DIM_H, DIM_H), lambda i: (0, 0)),
        ],
        out_specs=pl.BlockSpec((NC, BLK, HALF), lambda i: (0, i, 0)),
        out_shape=jax.ShapeDtypeStruct((NC, N_PAD, HALF), jnp.float32),
    )(p, y, dinv, b, W)


def _tc_tail(p, y, dinv, b, Wl, bl):
    def body(p_ref, y_ref, dinv_ref, b_ref, wl_ref, bl_ref, o_ref):
        agg = jnp.concatenate([p_ref[0], p_ref[1]], axis=-1)
        yv = jnp.concatenate([y_ref[0], y_ref[1]], axis=-1)
        dinv = dinv_ref[...]
        h = jnp.maximum(dinv * (agg + yv) + b_ref[...], 0.0)
        o_ref[...] = jnp.dot(h, wl_ref[...], preferred_element_type=jnp.float32) + bl_ref[...]

    return pl.pallas_call(
        body,
        grid=(GRID,),
        in_specs=[
            pl.BlockSpec((NC, BLK, HALF), lambda i: (0, i, 0)),
            pl.BlockSpec((NC, BLK, HALF), lambda i: (0, i, 0)),
            pl.BlockSpec((BLK, 1), lambda i: (i, 0)),
            pl.BlockSpec((1, DIM_H), lambda i: (0, 0)),
            pl.BlockSpec((DIM_H, N_Y), lambda i: (0, 0)),
            pl.BlockSpec((1, N_Y), lambda i: (0, 0)),
        ],
        out_specs=pl.BlockSpec((BLK, N_Y), lambda i: (i, 0)),
        out_shape=jax.ShapeDtypeStruct((N_PAD, N_Y), jnp.float32),
    )(p, y, dinv, b, Wl, bl)


def kernel(x, edge_index, batch, W1, b1, W2, b2, W3, b3, Wl, bl):
    del batch
    src = edge_index[0].astype(jnp.int32)
    dst = edge_index[1].astype(jnp.int32)
    pad = E_PAD - E
    src_r = jnp.concatenate([src, jnp.full((pad,), N, jnp.int32)]).reshape(NS, CHUNKS_T, CH)
    dst_r = jnp.concatenate([dst, jnp.zeros((pad,), jnp.int32)]).reshape(NS, CHUNKS_T, CH)
    emask = jnp.concatenate(
        [jnp.ones((E,), jnp.float32), jnp.zeros((pad,), jnp.float32)]
    ).reshape(NS, CHUNKS_T, CH)
    x_pad = jnp.pad(x, ((0, N_PAD - N), (0, 0)))

    deg_p = _sc_deg(dst_r, emask)
    y1, dinv = _tc_head(deg_p, x_pad, W1)
    p1 = _sc_agg(y1, src_r, dst_r)
    y2 = _tc_mid(p1, y1, dinv, b1.reshape(1, -1), W2)
    p2 = _sc_agg(y2, src_r, dst_r)
    y3 = _tc_mid(p2, y2, dinv, b2.reshape(1, -1), W3)
    p3 = _sc_agg(y3, src_r, dst_r)
    out = _tc_tail(p3, y3, dinv, b3.reshape(1, -1), Wl, bl.reshape(1, -1))
    return out[:N]


# speedup vs baseline: 40.7421x; 1.2684x over previous
import functools

import jax
import jax.numpy as jnp
from jax import lax
from jax.experimental import pallas as pl
from jax.experimental.pallas import tpu as pltpu
from jax.experimental.pallas import tpu_sc as plsc

N = 10000
E = 320000
N_X = 128
DIM_H = 64
N_Y = 10

NC, NS = 2, 16
HALF = DIM_H // NC
PK = 128 // HALF
CH = 128
CHUNKS_T = 160
E_PAD = NS * CHUNKS_T * CH
N_PAD = 10240
NROW = N_PAD // PK
ROWS_PER_TILE = N_PAD // NS

DCH = 80
DCHUNKS = E // NC // NS // DCH

BLK = 1024
BROW = BLK // PK
GRID = N_PAD // BLK

NBUF = 8
K_AHEAD = 4


def _mesh():
    return plsc.VectorSubcoreMesh(core_axis_name="c", subcore_axis_name="s")


_SC_PARAMS = dict(
    compiler_params=pltpu.CompilerParams(
        needs_layout_passes=False, use_tc_tiling_on_sc=False
    ),
)


def _sc_deg(dst_d):
    @functools.partial(
        pl.kernel,
        out_type=jax.ShapeDtypeStruct((NC, N_PAD, HALF), jnp.float32),
        mesh=_mesh(),
        scratch_types=[
            pltpu.VMEM((DCHUNKS, DCH), jnp.int32),
            pltpu.VMEM((N_PAD,), jnp.float32),
            pltpu.VMEM((NS, ROWS_PER_TILE), jnp.float32),
            pltpu.VMEM((ROWS_PER_TILE, HALF), jnp.float32),
            pltpu.VMEM_SHARED((NS, N_PAD), jnp.float32),
        ],
        **_SC_PARAMS,
    )
    def body(dst_hbm, out_hbm, idx_d, deg, slab, rep, stage_sp):
        c = lax.axis_index("c")
        s = lax.axis_index("s")
        pltpu.sync_copy(dst_hbm.at[c, s], idx_d)
        one16 = jnp.full((16,), 1.0, jnp.float32)
        zero16 = jnp.zeros((16,), jnp.float32)

        def zbody(i, carry):
            deg[pl.ds(i * 16, 16)] = zero16
            return carry

        lax.fori_loop(0, N_PAD // 16, zbody, 0)

        def cbody(j, carry):
            for g in range(DCH // 16):
                idx16 = idx_d[j, pl.ds(g * 16, 16)]
                plsc.addupdate_scatter(deg, [idx16], one16)
            return carry

        lax.fori_loop(0, DCHUNKS, cbody, 0)
        pltpu.sync_copy(deg, stage_sp.at[s])
        plsc.subcore_barrier()
        base = s * ROWS_PER_TILE
        pltpu.sync_copy(stage_sp.at[:, pl.ds(base, ROWS_PER_TILE)], slab)

        def sbody(m, carry):
            tot = slab[0, pl.ds(m * 16, 16)]
            for t in range(1, NS):
                tot = tot + slab[t, pl.ds(m * 16, 16)]
            deg[pl.ds(m * 16, 16)] = tot
            return carry

        lax.fori_loop(0, ROWS_PER_TILE // 16, sbody, 0)

        def rbody(m, carry):
            t = deg[pl.ds(m * 16, 16)]
            for l in range(16):
                v = jnp.full((16,), t[l], jnp.float32)
                for k in range(HALF // 16):
                    rep[m * 16 + l, pl.ds(k * 16, 16)] = v
            return carry

        lax.fori_loop(0, ROWS_PER_TILE // 16, rbody, 0)
        pltpu.sync_copy(rep, out_hbm.at[c, pl.ds(base, ROWS_PER_TILE)])

    return body(dst_d)


def _sc_agg(y, src_r, dst_r):
    @functools.partial(
        pl.kernel,
        out_type=jax.ShapeDtypeStruct((NC, N_PAD, HALF), jnp.float32),
        mesh=_mesh(),
        scratch_types=[
            pltpu.VMEM((CHUNKS_T, CH), jnp.int32),
            pltpu.VMEM((CHUNKS_T, CH), jnp.int32),
            pltpu.VMEM((NBUF, CH, HALF), jnp.float32),
            pltpu.VMEM_SHARED((N_PAD, HALF), jnp.float32),
            pltpu.VMEM_SHARED((N_PAD, HALF), jnp.float32),
            pltpu.SemaphoreType.DMA((NBUF,)),
            pltpu.SemaphoreType.DMA((NBUF,)),
        ],
        **_SC_PARAMS,
    )
    def body(y_hbm, src_hbm, dst_hbm, out_hbm, idx_s, idx_d, rows, acc, y_sp, sem_g, sem_s):
        c = lax.axis_index("c")
        s = lax.axis_index("s")
        pltpu.sync_copy(src_hbm.at[s], idx_s)
        pltpu.sync_copy(dst_hbm.at[s], idx_d)

        zero16 = jnp.zeros((16,), jnp.float32)

        def zbody(i, carry):
            for k in range(HALF // 16):
                rows[0, i, pl.ds(k * 16, 16)] = zero16
            return carry

        lax.fori_loop(0, CH, zbody, 0)
        base = s * ROWS_PER_TILE
        for k in range(ROWS_PER_TILE // CH):
            pltpu.sync_copy(rows.at[0], acc.at[pl.ds(base + k * CH, CH)])
        for k in range(ROWS_PER_TILE // CH):
            pltpu.sync_copy(y_hbm.at[c, pl.ds(base + k * CH, CH)], rows.at[1])
            pltpu.sync_copy(rows.at[1], y_sp.at[pl.ds(base + k * CH, CH)])
        plsc.subcore_barrier()

        for j in range(K_AHEAD):
            pltpu.async_copy(y_sp.at[idx_s.at[j]], rows.at[j], sem_g.at[j])

        def _wait(sem, b):
            pltpu.make_async_copy(y_hbm.at[0, pl.ds(0, CH)], rows.at[b], sem).wait()

        def cbody(j, carry):
            b = lax.rem(j, NBUF)
            _wait(sem_g.at[b], b)
            pltpu.async_copy(rows.at[b], acc.at[idx_d.at[j]], sem_s.at[b], add=True)
            jn = j + K_AHEAD
            bn = lax.rem(jn, NBUF)

            @pl.when(jn < CHUNKS_T)
            def _():
                @pl.when(jn >= NBUF)
                def _():
                    _wait(sem_s.at[bn], bn)
                pltpu.async_copy(y_sp.at[idx_s.at[jn]], rows.at[bn], sem_g.at[bn])

            return carry

        lax.fori_loop(0, CHUNKS_T, cbody, 0)
        for b in range(NBUF):
            _wait(sem_s.at[b], b)
        plsc.subcore_barrier()
        for k in range(ROWS_PER_TILE // CH):
            pltpu.sync_copy(acc.at[pl.ds(base + k * CH, CH)], rows.at[0])
            pltpu.sync_copy(rows.at[0], out_hbm.at[c, pl.ds(base + k * CH, CH)])

    return body(y, src_r, dst_r)


def _node_mask(i):
    k = lax.broadcasted_iota(jnp.int32, (BROW, 128), 0)
    q = lax.broadcasted_iota(jnp.int32, (BROW, 128), 1) // HALF
    node = i * BLK + PK * k + q
    return (node < N).astype(jnp.float32)


def _tc_head(deg_p, x_perm, W1):
    def body(deg_ref, x_ref, w_ref, y_ref, dinv_ref):
        i = pl.program_id(0)
        deg = deg_ref[0] + deg_ref[1] + _node_mask(i)
        dinv = jnp.where(deg > 0, lax.rsqrt(deg), 0.0)
        zs = [
            jnp.dot(x_ref[q], w_ref[...], preferred_element_type=jnp.float32)
            for q in range(PK)
        ]
        for c in range(NC):
            y_ref[c] = dinv * jnp.concatenate(
                [zs[q][:, c * HALF:(c + 1) * HALF] for q in range(PK)], axis=-1
            )
        dinv_ref[...] = dinv

    return pl.pallas_call(
        body,
        grid=(GRID,),
        in_specs=[
            pl.BlockSpec((NC, BROW, 128), lambda i: (0, i, 0)),
            pl.BlockSpec((PK, BROW, N_X), lambda i: (0, i, 0)),
            pl.BlockSpec((N_X, DIM_H), lambda i: (0, 0)),
        ],
        out_specs=[
            pl.BlockSpec((NC, BROW, 128), lambda i: (0, i, 0)),
            pl.BlockSpec((BROW, 128), lambda i: (i, 0)),
        ],
        out_shape=[
            jax.ShapeDtypeStruct((NC, NROW, 128), jnp.float32),
            jax.ShapeDtypeStruct((NROW, 128), jnp.float32),
        ],
    )(deg_p, x_perm, W1)


def _tc_mid(p, y, dinv, b_pack, BDW):
    def body(p_ref, y_ref, dinv_ref, b_ref, w_ref, o_ref):
        dinv = dinv_ref[...]
        hs = [
            jnp.maximum(dinv * (p_ref[c] + y_ref[c]) + b_ref[c], 0.0)
            for c in range(NC)
        ]
        for co in range(NC):
            z = sum(
                jnp.dot(hs[c], w_ref[c, co], preferred_element_type=jnp.float32)
                for c in range(NC)
            )
            o_ref[co] = z * dinv

    return pl.pallas_call(
        body,
        grid=(GRID,),
        in_specs=[
            pl.BlockSpec((NC, BROW, 128), lambda i: (0, i, 0)),
            pl.BlockSpec((NC, BROW, 128), lambda i: (0, i, 0)),
            pl.BlockSpec((BROW, 128), lambda i: (i, 0)),
            pl.BlockSpec((NC, 1, 128), lambda i: (0, 0, 0)),
            pl.BlockSpec((NC, NC, 128, 128), lambda i: (0, 0, 0, 0)),
        ],
        out_specs=pl.BlockSpec((NC, BROW, 128), lambda i: (0, i, 0)),
        out_shape=jax.ShapeDtypeStruct((NC, NROW, 128), jnp.float32),
    )(p, y, dinv, b_pack, BDW)


def _tc_tail(p, y, dinv, b_pack, BDWl, bl_pack):
    def body(p_ref, y_ref, dinv_ref, b_ref, wl_ref, bl_ref, o_ref):
        dinv = dinv_ref[...]
        z = bl_ref[...]
        for c in range(NC):
            h = jnp.maximum(dinv * (p_ref[c] + y_ref[c]) + b_ref[c], 0.0)
            z = z + jnp.dot(h, wl_ref[c], preferred_element_type=jnp.float32)
        o_ref[...] = z

    return pl.pallas_call(
        body,
        grid=(GRID,),
        in_specs=[
            pl.BlockSpec((NC, BROW, 128), lambda i: (0, i, 0)),
            pl.BlockSpec((NC, BROW, 128), lambda i: (0, i, 0)),
            pl.BlockSpec((BROW, 128), lambda i: (i, 0)),
            pl.BlockSpec((NC, 1, 128), lambda i: (0, 0, 0)),
            pl.BlockSpec((NC, 128, PK * N_Y), lambda i: (0, 0, 0)),
            pl.BlockSpec((1, PK * N_Y), lambda i: (0, 0)),
        ],
        out_specs=pl.BlockSpec((BROW, PK * N_Y), lambda i: (i, 0)),
        out_shape=jax.ShapeDtypeStruct((NROW, PK * N_Y), jnp.float32),
    )(p, y, dinv, b_pack, BDWl, bl_pack)


def _pack_b(b):
    return jnp.stack([jnp.tile(b[c * HALF:(c + 1) * HALF], PK)[None] for c in range(NC)])


def kernel(x, edge_index, batch, W1, b1, W2, b2, W3, b3, Wl, bl):
    del batch
    src = edge_index[0].astype(jnp.int32)
    dst = edge_index[1].astype(jnp.int32)
    pad = E_PAD - E
    src_r = jnp.concatenate([src, jnp.full((pad,), N, jnp.int32)]).reshape(NS, CHUNKS_T, CH)
    dst_r = jnp.concatenate([dst, jnp.zeros((pad,), jnp.int32)]).reshape(NS, CHUNKS_T, CH)
    dst_d = dst.reshape(NC, NS, DCHUNKS, DCH)
    x_perm = (
        jnp.pad(x, ((0, N_PAD - N), (0, 0))).reshape(NROW, PK, N_X).transpose(1, 0, 2)
    )
    eye = jnp.eye(PK, dtype=jnp.float32)
    BDW2 = jnp.stack([
        jnp.stack([jnp.kron(eye, W2[c * HALF:(c + 1) * HALF, co * HALF:(co + 1) * HALF])
                   for co in range(NC)]) for c in range(NC)])
    BDW3 = jnp.stack([
        jnp.stack([jnp.kron(eye, W3[c * HALF:(c + 1) * HALF, co * HALF:(co + 1) * HALF])
                   for co in range(NC)]) for c in range(NC)])
    BDWl = jnp.stack([jnp.kron(eye, Wl[c * HALF:(c + 1) * HALF]) for c in range(NC)])
    bl_pack = jnp.tile(bl, PK)[None]

    deg_p = _sc_deg(dst_d).reshape(NC, NROW, 128)
    y1, dinv = _tc_head(deg_p, x_perm, W1)
    p1 = _sc_agg(y1.reshape(NC, N_PAD, HALF), src_r, dst_r).reshape(NC, NROW, 128)
    y2 = _tc_mid(p1, y1, dinv, _pack_b(b1), BDW2)
    p2 = _sc_agg(y2.reshape(NC, N_PAD, HALF), src_r, dst_r).reshape(NC, NROW, 128)
    y3 = _tc_mid(p2, y2, dinv, _pack_b(b2), BDW3)
    p3 = _sc_agg(y3.reshape(NC, N_PAD, HALF), src_r, dst_r).reshape(NC, NROW, 128)
    out = _tc_tail(p3, y3, dinv, _pack_b(b3), BDWl, bl_pack)
    return out.reshape(N_PAD, N_Y)[:N]


# speedup vs baseline: 42.0060x; 1.0310x over previous
import functools

import jax
import jax.numpy as jnp
from jax import lax
from jax.experimental import pallas as pl
from jax.experimental.pallas import tpu as pltpu
from jax.experimental.pallas import tpu_sc as plsc

N = 10000
E = 320000
N_X = 128
DIM_H = 64
N_Y = 10

NC, NS = 2, 16
HALF = DIM_H // NC
PK = 128 // HALF
CH = 128
CHUNKS_T = 160
E_PAD = NS * CHUNKS_T * CH
N_PAD = 10240
NROW = N_PAD // PK
ROWS_PER_TILE = N_PAD // NS

DCH = 80
DCHUNKS = E // NC // NS // DCH

BLK = 1024
BROW = BLK // PK
GRID = N_PAD // BLK

NBUF = 12
K_AHEAD = 6


def _mesh():
    return plsc.VectorSubcoreMesh(core_axis_name="c", subcore_axis_name="s")


_SC_PARAMS = dict(
    compiler_params=pltpu.CompilerParams(
        needs_layout_passes=False, use_tc_tiling_on_sc=False
    ),
)


def _sc_deg(dst_d):
    @functools.partial(
        pl.kernel,
        out_type=jax.ShapeDtypeStruct((NC, N_PAD, HALF), jnp.float32),
        mesh=_mesh(),
        scratch_types=[
            pltpu.VMEM((DCHUNKS, DCH), jnp.int32),
            pltpu.VMEM((N_PAD,), jnp.float32),
            pltpu.VMEM((NS, ROWS_PER_TILE), jnp.float32),
            pltpu.VMEM((ROWS_PER_TILE, HALF), jnp.float32),
            pltpu.VMEM_SHARED((NS, N_PAD), jnp.float32),
        ],
        **_SC_PARAMS,
    )
    def body(dst_hbm, out_hbm, idx_d, deg, slab, rep, stage_sp):
        c = lax.axis_index("c")
        s = lax.axis_index("s")
        pltpu.sync_copy(dst_hbm.at[c, s], idx_d)
        one16 = jnp.full((16,), 1.0, jnp.float32)
        zero16 = jnp.zeros((16,), jnp.float32)

        def zbody(i, carry):
            deg[pl.ds(i * 16, 16)] = zero16
            return carry

        lax.fori_loop(0, N_PAD // 16, zbody, 0)

        def cbody(j, carry):
            for g in range(DCH // 16):
                idx16 = idx_d[j, pl.ds(g * 16, 16)]
                plsc.addupdate_scatter(deg, [idx16], one16)
            return carry

        lax.fori_loop(0, DCHUNKS, cbody, 0)
        pltpu.sync_copy(deg, stage_sp.at[s])
        plsc.subcore_barrier()
        base = s * ROWS_PER_TILE
        pltpu.sync_copy(stage_sp.at[:, pl.ds(base, ROWS_PER_TILE)], slab)

        def sbody(m, carry):
            tot = slab[0, pl.ds(m * 16, 16)]
            for t in range(1, NS):
                tot = tot + slab[t, pl.ds(m * 16, 16)]
            deg[pl.ds(m * 16, 16)] = tot
            return carry

        lax.fori_loop(0, ROWS_PER_TILE // 16, sbody, 0)

        def rbody(m, carry):
            t = deg[pl.ds(m * 16, 16)]
            for l in range(16):
                v = jnp.full((16,), t[l], jnp.float32)
                for k in range(HALF // 16):
                    rep[m * 16 + l, pl.ds(k * 16, 16)] = v
            return carry

        lax.fori_loop(0, ROWS_PER_TILE // 16, rbody, 0)
        pltpu.sync_copy(rep, out_hbm.at[c, pl.ds(base, ROWS_PER_TILE)])

    return body(dst_d)


def _sc_agg(y, src_r, dst_r):
    @functools.partial(
        pl.kernel,
        out_type=jax.ShapeDtypeStruct((NC, N_PAD, HALF), jnp.float32),
        mesh=_mesh(),
        scratch_types=[
            pltpu.VMEM((CHUNKS_T, CH), jnp.int32),
            pltpu.VMEM((CHUNKS_T, CH), jnp.int32),
            pltpu.VMEM((NBUF, CH, HALF), jnp.float32),
            pltpu.VMEM_SHARED((N_PAD, HALF), jnp.float32),
            pltpu.VMEM_SHARED((N_PAD, HALF), jnp.float32),
            pltpu.SemaphoreType.DMA((NBUF,)),
            pltpu.SemaphoreType.DMA((NBUF,)),
        ],
        **_SC_PARAMS,
    )
    def body(y_hbm, src_hbm, dst_hbm, out_hbm, idx_s, idx_d, rows, acc, y_sp, sem_g, sem_s):
        c = lax.axis_index("c")
        s = lax.axis_index("s")
        pltpu.sync_copy(src_hbm.at[s], idx_s)
        pltpu.sync_copy(dst_hbm.at[s], idx_d)

        zero16 = jnp.zeros((16,), jnp.float32)

        def zbody(i, carry):
            for k in range(HALF // 16):
                rows[0, i, pl.ds(k * 16, 16)] = zero16
            return carry

        lax.fori_loop(0, CH, zbody, 0)
        base = s * ROWS_PER_TILE
        NK = ROWS_PER_TILE // CH
        for k in range(NK):
            pltpu.async_copy(y_hbm.at[c, pl.ds(base + k * CH, CH)], rows.at[k + 1], sem_g.at[k])
            pltpu.async_copy(rows.at[0], acc.at[pl.ds(base + k * CH, CH)], sem_s.at[k])
        for k in range(NK):
            pltpu.make_async_copy(y_hbm.at[0, pl.ds(0, CH)], rows.at[k + 1], sem_g.at[k]).wait()
            pltpu.async_copy(rows.at[k + 1], y_sp.at[pl.ds(base + k * CH, CH)], sem_g.at[k])
        for k in range(NK):
            pltpu.make_async_copy(y_hbm.at[0, pl.ds(0, CH)], rows.at[k + 1], sem_g.at[k]).wait()
            pltpu.make_async_copy(y_hbm.at[0, pl.ds(0, CH)], rows.at[k + 1], sem_s.at[k]).wait()
        plsc.subcore_barrier()

        for j in range(K_AHEAD):
            pltpu.async_copy(y_sp.at[idx_s.at[j]], rows.at[j], sem_g.at[j])

        def _wait(sem, b):
            pltpu.make_async_copy(y_hbm.at[0, pl.ds(0, CH)], rows.at[b], sem).wait()

        def cbody(j, carry):
            b = lax.rem(j, NBUF)
            _wait(sem_g.at[b], b)
            pltpu.async_copy(rows.at[b], acc.at[idx_d.at[j]], sem_s.at[b], add=True)
            jn = j + K_AHEAD
            bn = lax.rem(jn, NBUF)

            @pl.when(jn < CHUNKS_T)
            def _():
                @pl.when(jn >= NBUF)
                def _():
                    _wait(sem_s.at[bn], bn)
                pltpu.async_copy(y_sp.at[idx_s.at[jn]], rows.at[bn], sem_g.at[bn])

            return carry

        lax.fori_loop(0, CHUNKS_T, cbody, 0)
        for b in range(NBUF):
            _wait(sem_s.at[b], b)
        plsc.subcore_barrier()
        for k in range(ROWS_PER_TILE // CH):
            pltpu.sync_copy(acc.at[pl.ds(base + k * CH, CH)], rows.at[0])
            pltpu.sync_copy(rows.at[0], out_hbm.at[c, pl.ds(base + k * CH, CH)])

    return body(y, src_r, dst_r)


def _node_mask(i):
    k = lax.broadcasted_iota(jnp.int32, (BROW, 128), 0)
    q = lax.broadcasted_iota(jnp.int32, (BROW, 128), 1) // HALF
    node = i * BLK + PK * k + q
    return (node < N).astype(jnp.float32)


def _tc_head(deg_p, x_perm, W1):
    def body(deg_ref, x_ref, w_ref, y_ref, dinv_ref):
        i = pl.program_id(0)
        deg = deg_ref[0] + deg_ref[1] + _node_mask(i)
        dinv = jnp.where(deg > 0, lax.rsqrt(deg), 0.0)
        zs = [
            jnp.dot(x_ref[q], w_ref[...], preferred_element_type=jnp.float32)
            for q in range(PK)
        ]
        for c in range(NC):
            y_ref[c] = dinv * jnp.concatenate(
                [zs[q][:, c * HALF:(c + 1) * HALF] for q in range(PK)], axis=-1
            )
        dinv_ref[...] = dinv

    return pl.pallas_call(
        body,
        grid=(GRID,),
        in_specs=[
            pl.BlockSpec((NC, BROW, 128), lambda i: (0, i, 0)),
            pl.BlockSpec((PK, BROW, N_X), lambda i: (0, i, 0)),
            pl.BlockSpec((N_X, DIM_H), lambda i: (0, 0)),
        ],
        out_specs=[
            pl.BlockSpec((NC, BROW, 128), lambda i: (0, i, 0)),
            pl.BlockSpec((BROW, 128), lambda i: (i, 0)),
        ],
        out_shape=[
            jax.ShapeDtypeStruct((NC, NROW, 128), jnp.float32),
            jax.ShapeDtypeStruct((NROW, 128), jnp.float32),
        ],
    )(deg_p, x_perm, W1)


def _tc_mid(p, y, dinv, b_pack, BDW):
    def body(p_ref, y_ref, dinv_ref, b_ref, w_ref, o_ref):
        dinv = dinv_ref[...]
        hs = [
            jnp.maximum(dinv * (p_ref[c] + y_ref[c]) + b_ref[c], 0.0)
            for c in range(NC)
        ]
        for co in range(NC):
            z = sum(
                jnp.dot(hs[c], w_ref[c, co], preferred_element_type=jnp.float32)
                for c in range(NC)
            )
            o_ref[co] = z * dinv

    return pl.pallas_call(
        body,
        grid=(GRID,),
        in_specs=[
            pl.BlockSpec((NC, BROW, 128), lambda i: (0, i, 0)),
            pl.BlockSpec((NC, BROW, 128), lambda i: (0, i, 0)),
            pl.BlockSpec((BROW, 128), lambda i: (i, 0)),
            pl.BlockSpec((NC, 1, 128), lambda i: (0, 0, 0)),
            pl.BlockSpec((NC, NC, 128, 128), lambda i: (0, 0, 0, 0)),
        ],
        out_specs=pl.BlockSpec((NC, BROW, 128), lambda i: (0, i, 0)),
        out_shape=jax.ShapeDtypeStruct((NC, NROW, 128), jnp.float32),
    )(p, y, dinv, b_pack, BDW)


def _tc_tail(p, y, dinv, b_pack, BDWl, bl_pack):
    def body(p_ref, y_ref, dinv_ref, b_ref, wl_ref, bl_ref, o_ref):
        dinv = dinv_ref[...]
        z = bl_ref[...]
        for c in range(NC):
            h = jnp.maximum(dinv * (p_ref[c] + y_ref[c]) + b_ref[c], 0.0)
            z = z + jnp.dot(h, wl_ref[c], preferred_element_type=jnp.float32)
        o_ref[...] = z

    return pl.pallas_call(
        body,
        grid=(GRID,),
        in_specs=[
            pl.BlockSpec((NC, BROW, 128), lambda i: (0, i, 0)),
            pl.BlockSpec((NC, BROW, 128), lambda i: (0, i, 0)),
            pl.BlockSpec((BROW, 128), lambda i: (i, 0)),
            pl.BlockSpec((NC, 1, 128), lambda i: (0, 0, 0)),
            pl.BlockSpec((NC, 128, PK * N_Y), lambda i: (0, 0, 0)),
            pl.BlockSpec((1, PK * N_Y), lambda i: (0, 0)),
        ],
        out_specs=pl.BlockSpec((BROW, PK * N_Y), lambda i: (i, 0)),
        out_shape=jax.ShapeDtypeStruct((NROW, PK * N_Y), jnp.float32),
    )(p, y, dinv, b_pack, BDWl, bl_pack)


def _pack_b(b):
    return jnp.stack([jnp.tile(b[c * HALF:(c + 1) * HALF], PK)[None] for c in range(NC)])


def kernel(x, edge_index, batch, W1, b1, W2, b2, W3, b3, Wl, bl):
    del batch
    src = edge_index[0].astype(jnp.int32)
    dst = edge_index[1].astype(jnp.int32)
    pad = E_PAD - E
    src_r = jnp.concatenate([src, jnp.full((pad,), N, jnp.int32)]).reshape(NS, CHUNKS_T, CH)
    dst_r = jnp.concatenate([dst, jnp.zeros((pad,), jnp.int32)]).reshape(NS, CHUNKS_T, CH)
    dst_d = dst.reshape(NC, NS, DCHUNKS, DCH)
    x_perm = (
        jnp.pad(x, ((0, N_PAD - N), (0, 0))).reshape(NROW, PK, N_X).transpose(1, 0, 2)
    )
    eye = jnp.eye(PK, dtype=jnp.float32)
    BDW2 = jnp.stack([
        jnp.stack([jnp.kron(eye, W2[c * HALF:(c + 1) * HALF, co * HALF:(co + 1) * HALF])
                   for co in range(NC)]) for c in range(NC)])
    BDW3 = jnp.stack([
        jnp.stack([jnp.kron(eye, W3[c * HALF:(c + 1) * HALF, co * HALF:(co + 1) * HALF])
                   for co in range(NC)]) for c in range(NC)])
    BDWl = jnp.stack([jnp.kron(eye, Wl[c * HALF:(c + 1) * HALF]) for c in range(NC)])
    bl_pack = jnp.tile(bl, PK)[None]

    deg_p = _sc_deg(dst_d).reshape(NC, NROW, 128)
    y1, dinv = _tc_head(deg_p, x_perm, W1)
    p1 = _sc_agg(y1.reshape(NC, N_PAD, HALF), src_r, dst_r).reshape(NC, NROW, 128)
    y2 = _tc_mid(p1, y1, dinv, _pack_b(b1), BDW2)
    p2 = _sc_agg(y2.reshape(NC, N_PAD, HALF), src_r, dst_r).reshape(NC, NROW, 128)
    y3 = _tc_mid(p2, y2, dinv, _pack_b(b2), BDW3)
    p3 = _sc_agg(y3.reshape(NC, N_PAD, HALF), src_r, dst_r).reshape(NC, NROW, 128)
    out = _tc_tail(p3, y3, dinv, _pack_b(b3), BDWl, bl_pack)
    return out.reshape(N_PAD, N_Y)[:N]


# speedup vs baseline: 43.7003x; 1.0403x over previous
import functools

import jax
import jax.numpy as jnp
from jax import lax
from jax.experimental import pallas as pl
from jax.experimental.pallas import tpu as pltpu
from jax.experimental.pallas import tpu_sc as plsc

N = 10000
E = 320000
N_X = 128
DIM_H = 64
N_Y = 10

NC, NS = 2, 16
HALF = DIM_H // NC
PK = 128 // HALF
CH = 128
FCHUNKS = E // (NS * CH)
TAIL = E - NS * FCHUNKS * CH
CHUNKS_T = FCHUNKS + 1
N_PAD = 10240
NROW = N_PAD // PK
ROWS_PER_TILE = N_PAD // NS

DCH = 80
DCHUNKS = E // NC // NS // DCH

BLK = 1024
BROW = BLK // PK
GRID = N_PAD // BLK

NBUF = 12
K_AHEAD = 6


def _mesh():
    return plsc.VectorSubcoreMesh(core_axis_name="c", subcore_axis_name="s")


_SC_PARAMS = dict(
    compiler_params=pltpu.CompilerParams(
        needs_layout_passes=False, use_tc_tiling_on_sc=False
    ),
)


def _sc_deg(dst_d):
    @functools.partial(
        pl.kernel,
        out_type=jax.ShapeDtypeStruct((NC, N_PAD, HALF), jnp.float32),
        mesh=_mesh(),
        scratch_types=[
            pltpu.VMEM((DCHUNKS, DCH), jnp.int32),
            pltpu.VMEM((N_PAD,), jnp.float32),
            pltpu.VMEM((NS, ROWS_PER_TILE), jnp.float32),
            pltpu.VMEM((ROWS_PER_TILE, HALF), jnp.float32),
            pltpu.VMEM_SHARED((NS, N_PAD), jnp.float32),
        ],
        **_SC_PARAMS,
    )
    def body(dst_hbm, out_hbm, idx_d, deg, slab, rep, stage_sp):
        c = lax.axis_index("c")
        s = lax.axis_index("s")
        pltpu.sync_copy(dst_hbm.at[c, s], idx_d)
        one16 = jnp.full((16,), 1.0, jnp.float32)
        zero16 = jnp.zeros((16,), jnp.float32)

        def zbody(i, carry):
            deg[pl.ds(i * 16, 16)] = zero16
            return carry

        lax.fori_loop(0, N_PAD // 16, zbody, 0)

        def cbody(j, carry):
            for g in range(DCH // 16):
                idx16 = idx_d[j, pl.ds(g * 16, 16)]
                plsc.addupdate_scatter(deg, [idx16], one16)
            return carry

        lax.fori_loop(0, DCHUNKS, cbody, 0)
        pltpu.sync_copy(deg, stage_sp.at[s])
        plsc.subcore_barrier()
        base = s * ROWS_PER_TILE
        pltpu.sync_copy(stage_sp.at[:, pl.ds(base, ROWS_PER_TILE)], slab)

        def sbody(m, carry):
            tot = slab[0, pl.ds(m * 16, 16)]
            for t in range(1, NS):
                tot = tot + slab[t, pl.ds(m * 16, 16)]
            deg[pl.ds(m * 16, 16)] = tot
            return carry

        lax.fori_loop(0, ROWS_PER_TILE // 16, sbody, 0)

        def rbody(m, carry):
            t = deg[pl.ds(m * 16, 16)]
            for l in range(16):
                v = jnp.full((16,), t[l], jnp.float32)
                for k in range(HALF // 16):
                    rep[m * 16 + l, pl.ds(k * 16, 16)] = v
            return carry

        lax.fori_loop(0, ROWS_PER_TILE // 16, rbody, 0)
        pltpu.sync_copy(rep, out_hbm.at[c, pl.ds(base, ROWS_PER_TILE)])

    return body(dst_d)


def _sc_agg(y, src_r, dst_r):
    @functools.partial(
        pl.kernel,
        out_type=jax.ShapeDtypeStruct((NC, N_PAD, HALF), jnp.float32),
        mesh=_mesh(),
        scratch_types=[
            pltpu.VMEM((CHUNKS_T, CH), jnp.int32),
            pltpu.VMEM((CHUNKS_T, CH), jnp.int32),
            pltpu.VMEM((NBUF, CH, HALF), jnp.float32),
            pltpu.VMEM_SHARED((N_PAD, HALF), jnp.float32),
            pltpu.VMEM_SHARED((N_PAD, HALF), jnp.float32),
            pltpu.SemaphoreType.DMA((NBUF,)),
            pltpu.SemaphoreType.DMA((NBUF,)),
        ],
        **_SC_PARAMS,
    )
    def body(y_hbm, srcA, srcB, dstA, dstB, out_hbm, idx_s, idx_d, rows, acc, y_sp, sem_g, sem_s):
        c = lax.axis_index("c")
        s = lax.axis_index("s")
        pltpu.sync_copy(srcA.at[s], idx_s.at[pl.ds(0, FCHUNKS)])
        pltpu.sync_copy(dstA.at[s], idx_d.at[pl.ds(0, FCHUNKS)])

        @pl.when(s < TAIL // CH)
        def _():
            pltpu.sync_copy(srcB.at[s], idx_s.at[FCHUNKS])
            pltpu.sync_copy(dstB.at[s], idx_d.at[FCHUNKS])

        @pl.when(s >= TAIL // CH)
        def _():
            padsrc = jnp.full((16,), N, jnp.int32)
            padzero = jnp.zeros((16,), jnp.int32)
            for g in range(CH // 16):
                idx_s[FCHUNKS, pl.ds(g * 16, 16)] = padsrc
                idx_d[FCHUNKS, pl.ds(g * 16, 16)] = padzero

        zero16 = jnp.zeros((16,), jnp.float32)

        def zbody(i, carry):
            for k in range(HALF // 16):
                rows[0, i, pl.ds(k * 16, 16)] = zero16
            return carry

        lax.fori_loop(0, CH, zbody, 0)
        base = s * ROWS_PER_TILE
        NK = ROWS_PER_TILE // CH
        for k in range(NK):
            pltpu.async_copy(y_hbm.at[c, pl.ds(base + k * CH, CH)], rows.at[k + 1], sem_g.at[k])
            pltpu.async_copy(rows.at[0], acc.at[pl.ds(base + k * CH, CH)], sem_s.at[k])
        for k in range(NK):
            pltpu.make_async_copy(y_hbm.at[0, pl.ds(0, CH)], rows.at[k + 1], sem_g.at[k]).wait()
            pltpu.async_copy(rows.at[k + 1], y_sp.at[pl.ds(base + k * CH, CH)], sem_g.at[k])
        for k in range(NK):
            pltpu.make_async_copy(y_hbm.at[0, pl.ds(0, CH)], rows.at[k + 1], sem_g.at[k]).wait()
            pltpu.make_async_copy(y_hbm.at[0, pl.ds(0, CH)], rows.at[k + 1], sem_s.at[k]).wait()
        plsc.subcore_barrier()

        for j in range(K_AHEAD):
            pltpu.async_copy(y_sp.at[idx_s.at[j]], rows.at[j], sem_g.at[j])

        def _wait(sem, b):
            pltpu.make_async_copy(y_hbm.at[0, pl.ds(0, CH)], rows.at[b], sem).wait()

        def cbody(j, carry):
            b = lax.rem(j, NBUF)
            _wait(sem_g.at[b], b)
            pltpu.async_copy(rows.at[b], acc.at[idx_d.at[j]], sem_s.at[b], add=True)
            jn = j + K_AHEAD
            bn = lax.rem(jn, NBUF)

            @pl.when(jn < CHUNKS_T)
            def _():
                @pl.when(jn >= NBUF)
                def _():
                    _wait(sem_s.at[bn], bn)
                pltpu.async_copy(y_sp.at[idx_s.at[jn]], rows.at[bn], sem_g.at[bn])

            return carry

        lax.fori_loop(0, CHUNKS_T, cbody, 0)
        for b in range(NBUF):
            _wait(sem_s.at[b], b)
        plsc.subcore_barrier()
        for k in range(ROWS_PER_TILE // CH):
            pltpu.sync_copy(acc.at[pl.ds(base + k * CH, CH)], rows.at[0])
            pltpu.sync_copy(rows.at[0], out_hbm.at[c, pl.ds(base + k * CH, CH)])

    return body(y, *src_r, *dst_r)


def _node_mask(i):
    k = lax.broadcasted_iota(jnp.int32, (BROW, 128), 0)
    q = lax.broadcasted_iota(jnp.int32, (BROW, 128), 1) // HALF
    node = i * BLK + PK * k + q
    return (node < N).astype(jnp.float32)


def _tc_head(deg_p, x_perm, W1):
    def body(deg_ref, x_ref, w_ref, y_ref, dinv_ref):
        i = pl.program_id(0)
        deg = deg_ref[0] + deg_ref[1] + _node_mask(i)
        dinv = jnp.where(deg > 0, lax.rsqrt(deg), 0.0)
        zs = [
            jnp.dot(x_ref[q], w_ref[...], preferred_element_type=jnp.float32)
            for q in range(PK)
        ]
        for c in range(NC):
            y_ref[c] = dinv * jnp.concatenate(
                [zs[q][:, c * HALF:(c + 1) * HALF] for q in range(PK)], axis=-1
            )
        dinv_ref[...] = dinv

    return pl.pallas_call(
        body,
        grid=(GRID,),
        in_specs=[
            pl.BlockSpec((NC, BROW, 128), lambda i: (0, i, 0)),
            pl.BlockSpec((PK, BROW, N_X), lambda i: (0, i, 0)),
            pl.BlockSpec((N_X, DIM_H), lambda i: (0, 0)),
        ],
        out_specs=[
            pl.BlockSpec((NC, BROW, 128), lambda i: (0, i, 0)),
            pl.BlockSpec((BROW, 128), lambda i: (i, 0)),
        ],
        out_shape=[
            jax.ShapeDtypeStruct((NC, NROW, 128), jnp.float32),
            jax.ShapeDtypeStruct((NROW, 128), jnp.float32),
        ],
    )(deg_p, x_perm, W1)


def _tc_mid(p, y, dinv, b_pack, BDW):
    def body(p_ref, y_ref, dinv_ref, b_ref, w_ref, o_ref):
        dinv = dinv_ref[...]
        hs = [
            jnp.maximum(dinv * (p_ref[c] + y_ref[c]) + b_ref[c], 0.0)
            for c in range(NC)
        ]
        for co in range(NC):
            z = sum(
                jnp.dot(hs[c], w_ref[c, co], preferred_element_type=jnp.float32)
                for c in range(NC)
            )
            o_ref[co] = z * dinv

    return pl.pallas_call(
        body,
        grid=(GRID,),
        in_specs=[
            pl.BlockSpec((NC, BROW, 128), lambda i: (0, i, 0)),
            pl.BlockSpec((NC, BROW, 128), lambda i: (0, i, 0)),
            pl.BlockSpec((BROW, 128), lambda i: (i, 0)),
            pl.BlockSpec((NC, 1, 128), lambda i: (0, 0, 0)),
            pl.BlockSpec((NC, NC, 128, 128), lambda i: (0, 0, 0, 0)),
        ],
        out_specs=pl.BlockSpec((NC, BROW, 128), lambda i: (0, i, 0)),
        out_shape=jax.ShapeDtypeStruct((NC, NROW, 128), jnp.float32),
    )(p, y, dinv, b_pack, BDW)


def _tc_tail(p, y, dinv, b_pack, BDWl, bl_pack):
    def body(p_ref, y_ref, dinv_ref, b_ref, wl_ref, bl_ref, o_ref):
        dinv = dinv_ref[...]
        z = bl_ref[...]
        for c in range(NC):
            h = jnp.maximum(dinv * (p_ref[c] + y_ref[c]) + b_ref[c], 0.0)
            z = z + jnp.dot(h, wl_ref[c], preferred_element_type=jnp.float32)
        o_ref[...] = z

    return pl.pallas_call(
        body,
        grid=(GRID,),
        in_specs=[
            pl.BlockSpec((NC, BROW, 128), lambda i: (0, i, 0)),
            pl.BlockSpec((NC, BROW, 128), lambda i: (0, i, 0)),
            pl.BlockSpec((BROW, 128), lambda i: (i, 0)),
            pl.BlockSpec((NC, 1, 128), lambda i: (0, 0, 0)),
            pl.BlockSpec((NC, 128, PK * N_Y), lambda i: (0, 0, 0)),
            pl.BlockSpec((1, PK * N_Y), lambda i: (0, 0)),
        ],
        out_specs=pl.BlockSpec((BROW, PK * N_Y), lambda i: (i, 0)),
        out_shape=jax.ShapeDtypeStruct((NROW, PK * N_Y), jnp.float32),
    )(p, y, dinv, b_pack, BDWl, bl_pack)


def _pack_b(b):
    return jnp.stack([jnp.tile(b[c * HALF:(c + 1) * HALF], PK)[None] for c in range(NC)])


def kernel(x, edge_index, batch, W1, b1, W2, b2, W3, b3, Wl, bl):
    del batch
    src = edge_index[0].astype(jnp.int32)
    dst = edge_index[1].astype(jnp.int32)
    nfull = NS * FCHUNKS * CH
    src_r = (src[:nfull].reshape(NS, FCHUNKS, CH), src[nfull:].reshape(TAIL // CH, CH))
    dst_r = (dst[:nfull].reshape(NS, FCHUNKS, CH), dst[nfull:].reshape(TAIL // CH, CH))
    dst_d = dst.reshape(NC, NS, DCHUNKS, DCH)
    x_perm = (
        jnp.pad(x, ((0, N_PAD - N), (0, 0))).reshape(NROW, PK, N_X).transpose(1, 0, 2)
    )
    eye = jnp.eye(PK, dtype=jnp.float32)
    BDW2 = jnp.stack([
        jnp.stack([jnp.kron(eye, W2[c * HALF:(c + 1) * HALF, co * HALF:(co + 1) * HALF])
                   for co in range(NC)]) for c in range(NC)])
    BDW3 = jnp.stack([
        jnp.stack([jnp.kron(eye, W3[c * HALF:(c + 1) * HALF, co * HALF:(co + 1) * HALF])
                   for co in range(NC)]) for c in range(NC)])
    BDWl = jnp.stack([jnp.kron(eye, Wl[c * HALF:(c + 1) * HALF]) for c in range(NC)])
    bl_pack = jnp.tile(bl, PK)[None]

    deg_p = _sc_deg(dst_d).reshape(NC, NROW, 128)
    y1, dinv = _tc_head(deg_p, x_perm, W1)
    p1 = _sc_agg(y1.reshape(NC, N_PAD, HALF), src_r, dst_r).reshape(NC, NROW, 128)
    y2 = _tc_mid(p1, y1, dinv, _pack_b(b1), BDW2)
    p2 = _sc_agg(y2.reshape(NC, N_PAD, HALF), src_r, dst_r).reshape(NC, NROW, 128)
    y3 = _tc_mid(p2, y2, dinv, _pack_b(b2), BDW3)
    p3 = _sc_agg(y3.reshape(NC, N_PAD, HALF), src_r, dst_r).reshape(NC, NROW, 128)
    out = _tc_tail(p3, y3, dinv, _pack_b(b3), BDWl, bl_pack)
    return out.reshape(N_PAD, N_Y)[:N]


# speedup vs baseline: 45.3011x; 1.0366x over previous
import functools

import jax
import jax.numpy as jnp
from jax import lax
from jax.experimental import pallas as pl
from jax.experimental.pallas import tpu as pltpu
from jax.experimental.pallas import tpu_sc as plsc

N = 10000
E = 320000
N_X = 128
DIM_H = 64
N_Y = 10

NC, NS = 2, 16
HALF = DIM_H // NC
PK = 128 // HALF
CH = 128
FCHUNKS = E // (NS * CH)
TAIL = E - NS * FCHUNKS * CH
CHUNKS_T = FCHUNKS + 1
N_PAD = 10240
NROW = N_PAD // PK
ROWS_PER_TILE = N_PAD // NS

DCH = 80
DCHUNKS = E // NC // NS // DCH

BLK = 2048
BROW = BLK // PK
GRID = N_PAD // BLK

NBUF = 12
K_AHEAD = 6


def _mesh():
    return plsc.VectorSubcoreMesh(core_axis_name="c", subcore_axis_name="s")


_SC_PARAMS = dict(
    compiler_params=pltpu.CompilerParams(
        needs_layout_passes=False, use_tc_tiling_on_sc=False
    ),
)


def _sc_deg(dst_d):
    @functools.partial(
        pl.kernel,
        out_type=jax.ShapeDtypeStruct((NC, N_PAD, HALF), jnp.float32),
        mesh=_mesh(),
        scratch_types=[
            pltpu.VMEM((DCHUNKS, DCH), jnp.int32),
            pltpu.VMEM((N_PAD,), jnp.float32),
            pltpu.VMEM((NS, ROWS_PER_TILE), jnp.float32),
            pltpu.VMEM((ROWS_PER_TILE, HALF), jnp.float32),
            pltpu.VMEM_SHARED((NS, N_PAD), jnp.float32),
        ],
        **_SC_PARAMS,
    )
    def body(dst_hbm, out_hbm, idx_d, deg, slab, rep, stage_sp):
        c = lax.axis_index("c")
        s = lax.axis_index("s")
        pltpu.sync_copy(dst_hbm.at[c, s], idx_d)
        one16 = jnp.full((16,), 1.0, jnp.float32)
        zero16 = jnp.zeros((16,), jnp.float32)

        def zbody(i, carry):
            deg[pl.ds(i * 16, 16)] = zero16
            return carry

        lax.fori_loop(0, N_PAD // 16, zbody, 0)

        def cbody(j, carry):
            for g in range(DCH // 16):
                idx16 = idx_d[j, pl.ds(g * 16, 16)]
                plsc.addupdate_scatter(deg, [idx16], one16)
            return carry

        lax.fori_loop(0, DCHUNKS, cbody, 0)
        pltpu.sync_copy(deg, stage_sp.at[s])
        plsc.subcore_barrier()
        base = s * ROWS_PER_TILE
        pltpu.sync_copy(stage_sp.at[:, pl.ds(base, ROWS_PER_TILE)], slab)

        def sbody(m, carry):
            tot = slab[0, pl.ds(m * 16, 16)]
            for t in range(1, NS):
                tot = tot + slab[t, pl.ds(m * 16, 16)]
            deg[pl.ds(m * 16, 16)] = tot
            return carry

        lax.fori_loop(0, ROWS_PER_TILE // 16, sbody, 0)

        def rbody(m, carry):
            t = deg[pl.ds(m * 16, 16)]
            for l in range(16):
                v = jnp.full((16,), t[l], jnp.float32)
                for k in range(HALF // 16):
                    rep[m * 16 + l, pl.ds(k * 16, 16)] = v
            return carry

        lax.fori_loop(0, ROWS_PER_TILE // 16, rbody, 0)
        pltpu.sync_copy(rep, out_hbm.at[c, pl.ds(base, ROWS_PER_TILE)])

    return body(dst_d)


def _sc_agg(y, src_r, dst_r):
    @functools.partial(
        pl.kernel,
        out_type=jax.ShapeDtypeStruct((NC, N_PAD, HALF), jnp.float32),
        mesh=_mesh(),
        scratch_types=[
            pltpu.VMEM((CHUNKS_T, CH), jnp.int32),
            pltpu.VMEM((CHUNKS_T, CH), jnp.int32),
            pltpu.VMEM((NBUF, CH, HALF), jnp.float32),
            pltpu.VMEM_SHARED((N_PAD, HALF), jnp.float32),
            pltpu.VMEM_SHARED((N_PAD, HALF), jnp.float32),
            pltpu.SemaphoreType.DMA((NBUF,)),
            pltpu.SemaphoreType.DMA((NBUF,)),
        ],
        **_SC_PARAMS,
    )
    def body(y_hbm, srcA, srcB, dstA, dstB, out_hbm, idx_s, idx_d, rows, acc, y_sp, sem_g, sem_s):
        c = lax.axis_index("c")
        s = lax.axis_index("s")
        pltpu.sync_copy(srcA.at[s], idx_s.at[pl.ds(0, FCHUNKS)])
        pltpu.sync_copy(dstA.at[s], idx_d.at[pl.ds(0, FCHUNKS)])

        @pl.when(s < TAIL // CH)
        def _():
            pltpu.sync_copy(srcB.at[s], idx_s.at[FCHUNKS])
            pltpu.sync_copy(dstB.at[s], idx_d.at[FCHUNKS])

        @pl.when(s >= TAIL // CH)
        def _():
            padsrc = jnp.full((16,), N, jnp.int32)
            padzero = jnp.zeros((16,), jnp.int32)
            for g in range(CH // 16):
                idx_s[FCHUNKS, pl.ds(g * 16, 16)] = padsrc
                idx_d[FCHUNKS, pl.ds(g * 16, 16)] = padzero

        zero16 = jnp.zeros((16,), jnp.float32)

        def zbody(i, carry):
            for k in range(HALF // 16):
                rows[0, i, pl.ds(k * 16, 16)] = zero16
            return carry

        lax.fori_loop(0, CH, zbody, 0)
        base = s * ROWS_PER_TILE
        NK = ROWS_PER_TILE // CH
        for k in range(NK):
            pltpu.async_copy(y_hbm.at[c, pl.ds(base + k * CH, CH)], rows.at[k + 1], sem_g.at[k])
            pltpu.async_copy(rows.at[0], acc.at[pl.ds(base + k * CH, CH)], sem_s.at[k])
        for k in range(NK):
            pltpu.make_async_copy(y_hbm.at[0, pl.ds(0, CH)], rows.at[k + 1], sem_g.at[k]).wait()
            pltpu.async_copy(rows.at[k + 1], y_sp.at[pl.ds(base + k * CH, CH)], sem_g.at[k])
        for k in range(NK):
            pltpu.make_async_copy(y_hbm.at[0, pl.ds(0, CH)], rows.at[k + 1], sem_g.at[k]).wait()
            pltpu.make_async_copy(y_hbm.at[0, pl.ds(0, CH)], rows.at[k + 1], sem_s.at[k]).wait()
        plsc.subcore_barrier()

        for j in range(K_AHEAD):
            pltpu.async_copy(y_sp.at[idx_s.at[j]], rows.at[j], sem_g.at[j])

        def _wait(sem, b):
            pltpu.make_async_copy(y_hbm.at[0, pl.ds(0, CH)], rows.at[b], sem).wait()

        def cbody(j, carry):
            b = lax.rem(j, NBUF)
            _wait(sem_g.at[b], b)
            pltpu.async_copy(rows.at[b], acc.at[idx_d.at[j]], sem_s.at[b], add=True)
            jn = j + K_AHEAD
            bn = lax.rem(jn, NBUF)

            @pl.when(jn < CHUNKS_T)
            def _():
                @pl.when(jn >= NBUF)
                def _():
                    _wait(sem_s.at[bn], bn)
                pltpu.async_copy(y_sp.at[idx_s.at[jn]], rows.at[bn], sem_g.at[bn])

            return carry

        lax.fori_loop(0, CHUNKS_T, cbody, 0)
        for b in range(NBUF):
            _wait(sem_s.at[b], b)
        plsc.subcore_barrier()
        for k in range(ROWS_PER_TILE // CH):
            pltpu.sync_copy(acc.at[pl.ds(base + k * CH, CH)], rows.at[0])
            pltpu.sync_copy(rows.at[0], out_hbm.at[c, pl.ds(base + k * CH, CH)])

    return body(y, *src_r, *dst_r)


def _node_mask(i):
    k = lax.broadcasted_iota(jnp.int32, (BROW, 128), 0)
    q = lax.broadcasted_iota(jnp.int32, (BROW, 128), 1) // HALF
    node = i * BLK + PK * k + q
    return (node < N).astype(jnp.float32)


def _tc_head(deg_p, x_perm, W1):
    def body(deg_ref, x_ref, w_ref, y_ref, dinv_ref):
        i = pl.program_id(0)
        deg = deg_ref[0] + deg_ref[1] + _node_mask(i)
        dinv = jnp.where(deg > 0, lax.rsqrt(deg), 0.0)
        zs = [
            jnp.dot(x_ref[q], w_ref[...], preferred_element_type=jnp.float32)
            for q in range(PK)
        ]
        for c in range(NC):
            y_ref[c] = dinv * jnp.concatenate(
                [zs[q][:, c * HALF:(c + 1) * HALF] for q in range(PK)], axis=-1
            )
        dinv_ref[...] = dinv

    return pl.pallas_call(
        body,
        grid=(GRID,),
        in_specs=[
            pl.BlockSpec((NC, BROW, 128), lambda i: (0, i, 0)),
            pl.BlockSpec((PK, BROW, N_X), lambda i: (0, i, 0)),
            pl.BlockSpec((N_X, DIM_H), lambda i: (0, 0)),
        ],
        out_specs=[
            pl.BlockSpec((NC, BROW, 128), lambda i: (0, i, 0)),
            pl.BlockSpec((BROW, 128), lambda i: (i, 0)),
        ],
        out_shape=[
            jax.ShapeDtypeStruct((NC, NROW, 128), jnp.float32),
            jax.ShapeDtypeStruct((NROW, 128), jnp.float32),
        ],
    )(deg_p, x_perm, W1)


def _tc_mid(p, y, dinv, b_pack, BDW):
    def body(p_ref, y_ref, dinv_ref, b_ref, w_ref, o_ref):
        dinv = dinv_ref[...]
        hs = [
            jnp.maximum(dinv * (p_ref[c] + y_ref[c]) + b_ref[c], 0.0)
            for c in range(NC)
        ]
        for co in range(NC):
            z = sum(
                jnp.dot(hs[c], w_ref[c, co], preferred_element_type=jnp.float32)
                for c in range(NC)
            )
            o_ref[co] = z * dinv

    return pl.pallas_call(
        body,
        grid=(GRID,),
        in_specs=[
            pl.BlockSpec((NC, BROW, 128), lambda i: (0, i, 0)),
            pl.BlockSpec((NC, BROW, 128), lambda i: (0, i, 0)),
            pl.BlockSpec((BROW, 128), lambda i: (i, 0)),
            pl.BlockSpec((NC, 1, 128), lambda i: (0, 0, 0)),
            pl.BlockSpec((NC, NC, 128, 128), lambda i: (0, 0, 0, 0)),
        ],
        out_specs=pl.BlockSpec((NC, BROW, 128), lambda i: (0, i, 0)),
        out_shape=jax.ShapeDtypeStruct((NC, NROW, 128), jnp.float32),
    )(p, y, dinv, b_pack, BDW)


def _tc_tail(p, y, dinv, b_pack, BDWl, bl_pack):
    def body(p_ref, y_ref, dinv_ref, b_ref, wl_ref, bl_ref, o_ref):
        dinv = dinv_ref[...]
        z = bl_ref[...]
        for c in range(NC):
            h = jnp.maximum(dinv * (p_ref[c] + y_ref[c]) + b_ref[c], 0.0)
            z = z + jnp.dot(h, wl_ref[c], preferred_element_type=jnp.float32)
        o_ref[...] = z

    return pl.pallas_call(
        body,
        grid=(GRID,),
        in_specs=[
            pl.BlockSpec((NC, BROW, 128), lambda i: (0, i, 0)),
            pl.BlockSpec((NC, BROW, 128), lambda i: (0, i, 0)),
            pl.BlockSpec((BROW, 128), lambda i: (i, 0)),
            pl.BlockSpec((NC, 1, 128), lambda i: (0, 0, 0)),
            pl.BlockSpec((NC, 128, PK * N_Y), lambda i: (0, 0, 0)),
            pl.BlockSpec((1, PK * N_Y), lambda i: (0, 0)),
        ],
        out_specs=pl.BlockSpec((BROW, PK * N_Y), lambda i: (i, 0)),
        out_shape=jax.ShapeDtypeStruct((NROW, PK * N_Y), jnp.float32),
    )(p, y, dinv, b_pack, BDWl, bl_pack)


def _pack_b(b):
    return jnp.stack([jnp.tile(b[c * HALF:(c + 1) * HALF], PK)[None] for c in range(NC)])


def kernel(x, edge_index, batch, W1, b1, W2, b2, W3, b3, Wl, bl):
    del batch
    src = edge_index[0].astype(jnp.int32)
    dst = edge_index[1].astype(jnp.int32)
    nfull = NS * FCHUNKS * CH
    src_r = (src[:nfull].reshape(NS, FCHUNKS, CH), src[nfull:].reshape(TAIL // CH, CH))
    dst_r = (dst[:nfull].reshape(NS, FCHUNKS, CH), dst[nfull:].reshape(TAIL // CH, CH))
    dst_d = dst.reshape(NC, NS, DCHUNKS, DCH)
    x_perm = (
        jnp.pad(x, ((0, N_PAD - N), (0, 0))).reshape(NROW, PK, N_X).transpose(1, 0, 2)
    )
    eye = jnp.eye(PK, dtype=jnp.float32)
    BDW2 = jnp.stack([
        jnp.stack([jnp.kron(eye, W2[c * HALF:(c + 1) * HALF, co * HALF:(co + 1) * HALF])
                   for co in range(NC)]) for c in range(NC)])
    BDW3 = jnp.stack([
        jnp.stack([jnp.kron(eye, W3[c * HALF:(c + 1) * HALF, co * HALF:(co + 1) * HALF])
                   for co in range(NC)]) for c in range(NC)])
    BDWl = jnp.stack([jnp.kron(eye, Wl[c * HALF:(c + 1) * HALF]) for c in range(NC)])
    bl_pack = jnp.tile(bl, PK)[None]

    deg_p = _sc_deg(dst_d).reshape(NC, NROW, 128)
    y1, dinv = _tc_head(deg_p, x_perm, W1)
    p1 = _sc_agg(y1.reshape(NC, N_PAD, HALF), src_r, dst_r).reshape(NC, NROW, 128)
    y2 = _tc_mid(p1, y1, dinv, _pack_b(b1), BDW2)
    p2 = _sc_agg(y2.reshape(NC, N_PAD, HALF), src_r, dst_r).reshape(NC, NROW, 128)
    y3 = _tc_mid(p2, y2, dinv, _pack_b(b2), BDW3)
    p3 = _sc_agg(y3.reshape(NC, N_PAD, HALF), src_r, dst_r).reshape(NC, NROW, 128)
    out = _tc_tail(p3, y3, dinv, _pack_b(b3), BDWl, bl_pack)
    return out.reshape(N_PAD, N_Y)[:N]


# speedup vs baseline: 46.8044x; 1.0332x over previous
import functools

import jax
import jax.numpy as jnp
from jax import lax
from jax.experimental import pallas as pl
from jax.experimental.pallas import tpu as pltpu
from jax.experimental.pallas import tpu_sc as plsc

N = 10000
E = 320000
N_X = 128
DIM_H = 64
N_Y = 10

NC, NS = 2, 16
HALF = DIM_H // NC
PK = 128 // HALF
CH = 128
FCHUNKS = E // (NS * CH)
TAIL = E - NS * FCHUNKS * CH
CHUNKS_T = FCHUNKS + 1
N_PAD = 10240
NROW = N_PAD // PK
ROWS_PER_TILE = N_PAD // NS

DCH = 80
DCHUNKS = E // NC // NS // DCH

BLK = 5120
BROW = BLK // PK
GRID = N_PAD // BLK

NBUF = 12
K_AHEAD = 6


def _mesh():
    return plsc.VectorSubcoreMesh(core_axis_name="c", subcore_axis_name="s")


_SC_PARAMS = dict(
    compiler_params=pltpu.CompilerParams(
        needs_layout_passes=False, use_tc_tiling_on_sc=False
    ),
)


def _sc_deg(dst_d):
    @functools.partial(
        pl.kernel,
        out_type=jax.ShapeDtypeStruct((NC, N_PAD, HALF), jnp.float32),
        mesh=_mesh(),
        scratch_types=[
            pltpu.VMEM((DCHUNKS, DCH), jnp.int32),
            pltpu.VMEM((N_PAD,), jnp.float32),
            pltpu.VMEM((NS, ROWS_PER_TILE), jnp.float32),
            pltpu.VMEM((ROWS_PER_TILE, HALF), jnp.float32),
            pltpu.VMEM_SHARED((NS, N_PAD), jnp.float32),
        ],
        **_SC_PARAMS,
    )
    def body(dst_hbm, out_hbm, idx_d, deg, slab, rep, stage_sp):
        c = lax.axis_index("c")
        s = lax.axis_index("s")
        pltpu.sync_copy(dst_hbm.at[c, s], idx_d)
        one16 = jnp.full((16,), 1.0, jnp.float32)
        zero16 = jnp.zeros((16,), jnp.float32)

        def zbody(i, carry):
            deg[pl.ds(i * 16, 16)] = zero16
            return carry

        lax.fori_loop(0, N_PAD // 16, zbody, 0)

        def cbody(j, carry):
            for g in range(DCH // 16):
                idx16 = idx_d[j, pl.ds(g * 16, 16)]
                plsc.addupdate_scatter(deg, [idx16], one16)
            return carry

        lax.fori_loop(0, DCHUNKS, cbody, 0)
        pltpu.sync_copy(deg, stage_sp.at[s])
        plsc.subcore_barrier()
        base = s * ROWS_PER_TILE
        pltpu.sync_copy(stage_sp.at[:, pl.ds(base, ROWS_PER_TILE)], slab)

        def sbody(m, carry):
            tot = slab[0, pl.ds(m * 16, 16)]
            for t in range(1, NS):
                tot = tot + slab[t, pl.ds(m * 16, 16)]
            deg[pl.ds(m * 16, 16)] = tot
            return carry

        lax.fori_loop(0, ROWS_PER_TILE // 16, sbody, 0)

        def rbody(m, carry):
            t = deg[pl.ds(m * 16, 16)]
            for l in range(16):
                v = jnp.full((16,), t[l], jnp.float32)
                for k in range(HALF // 16):
                    rep[m * 16 + l, pl.ds(k * 16, 16)] = v
            return carry

        lax.fori_loop(0, ROWS_PER_TILE // 16, rbody, 0)
        pltpu.sync_copy(rep, out_hbm.at[c, pl.ds(base, ROWS_PER_TILE)])

    return body(dst_d)


def _sc_agg(y, src_r, dst_r):
    @functools.partial(
        pl.kernel,
        out_type=jax.ShapeDtypeStruct((NC, N_PAD, HALF), jnp.float32),
        mesh=_mesh(),
        scratch_types=[
            pltpu.VMEM((CHUNKS_T, CH), jnp.int32),
            pltpu.VMEM((CHUNKS_T, CH), jnp.int32),
            pltpu.VMEM((NBUF, CH, HALF), jnp.float32),
            pltpu.VMEM_SHARED((N_PAD, HALF), jnp.float32),
            pltpu.VMEM_SHARED((N_PAD, HALF), jnp.float32),
            pltpu.SemaphoreType.DMA((NBUF,)),
            pltpu.SemaphoreType.DMA((NBUF,)),
        ],
        **_SC_PARAMS,
    )
    def body(y_hbm, srcA, srcB, dstA, dstB, out_hbm, idx_s, idx_d, rows, acc, y_sp, sem_g, sem_s):
        c = lax.axis_index("c")
        s = lax.axis_index("s")
        pltpu.sync_copy(srcA.at[s], idx_s.at[pl.ds(0, FCHUNKS)])
        pltpu.sync_copy(dstA.at[s], idx_d.at[pl.ds(0, FCHUNKS)])

        @pl.when(s < TAIL // CH)
        def _():
            pltpu.sync_copy(srcB.at[s], idx_s.at[FCHUNKS])
            pltpu.sync_copy(dstB.at[s], idx_d.at[FCHUNKS])

        @pl.when(s >= TAIL // CH)
        def _():
            padsrc = jnp.full((16,), N, jnp.int32)
            padzero = jnp.zeros((16,), jnp.int32)
            for g in range(CH // 16):
                idx_s[FCHUNKS, pl.ds(g * 16, 16)] = padsrc
                idx_d[FCHUNKS, pl.ds(g * 16, 16)] = padzero

        zero16 = jnp.zeros((16,), jnp.float32)

        def zbody(i, carry):
            for k in range(HALF // 16):
                rows[0, i, pl.ds(k * 16, 16)] = zero16
            return carry

        lax.fori_loop(0, CH, zbody, 0)
        base = s * ROWS_PER_TILE
        NK = ROWS_PER_TILE // CH
        for k in range(NK):
            pltpu.async_copy(y_hbm.at[c, pl.ds(base + k * CH, CH)], rows.at[k + 1], sem_g.at[k])
            pltpu.async_copy(rows.at[0], acc.at[pl.ds(base + k * CH, CH)], sem_s.at[k])
        for k in range(NK):
            pltpu.make_async_copy(y_hbm.at[0, pl.ds(0, CH)], rows.at[k + 1], sem_g.at[k]).wait()
            pltpu.async_copy(rows.at[k + 1], y_sp.at[pl.ds(base + k * CH, CH)], sem_g.at[k])
        for k in range(NK):
            pltpu.make_async_copy(y_hbm.at[0, pl.ds(0, CH)], rows.at[k + 1], sem_g.at[k]).wait()
            pltpu.make_async_copy(y_hbm.at[0, pl.ds(0, CH)], rows.at[k + 1], sem_s.at[k]).wait()
        plsc.subcore_barrier()

        for j in range(K_AHEAD):
            pltpu.async_copy(y_sp.at[idx_s.at[j]], rows.at[j], sem_g.at[j])

        def _wait(sem, b):
            pltpu.make_async_copy(y_hbm.at[0, pl.ds(0, CH)], rows.at[b], sem).wait()

        def cbody(j, carry):
            b = lax.rem(j, NBUF)
            _wait(sem_g.at[b], b)
            pltpu.async_copy(rows.at[b], acc.at[idx_d.at[j]], sem_s.at[b], add=True)
            jn = j + K_AHEAD
            bn = lax.rem(jn, NBUF)

            @pl.when(jn < CHUNKS_T)
            def _():
                @pl.when(jn >= NBUF)
                def _():
                    _wait(sem_s.at[bn], bn)
                pltpu.async_copy(y_sp.at[idx_s.at[jn]], rows.at[bn], sem_g.at[bn])

            return carry

        lax.fori_loop(0, CHUNKS_T, cbody, 0)
        for b in range(NBUF):
            _wait(sem_s.at[b], b)
        plsc.subcore_barrier()
        for k in range(ROWS_PER_TILE // CH):
            pltpu.sync_copy(acc.at[pl.ds(base + k * CH, CH)], rows.at[0])
            pltpu.sync_copy(rows.at[0], out_hbm.at[c, pl.ds(base + k * CH, CH)])

    return body(y, *src_r, *dst_r)


def _node_mask(i):
    k = lax.broadcasted_iota(jnp.int32, (BROW, 128), 0)
    q = lax.broadcasted_iota(jnp.int32, (BROW, 128), 1) // HALF
    node = i * BLK + PK * k + q
    return (node < N).astype(jnp.float32)


def _tc_head(deg_p, x_perm, W1):
    def body(deg_ref, x_ref, w_ref, y_ref, dinv_ref):
        i = pl.program_id(0)
        deg = deg_ref[0] + deg_ref[1] + _node_mask(i)
        dinv = jnp.where(deg > 0, lax.rsqrt(deg), 0.0)
        zs = [
            jnp.dot(x_ref[q], w_ref[...], preferred_element_type=jnp.float32)
            for q in range(PK)
        ]
        for c in range(NC):
            y_ref[c] = dinv * jnp.concatenate(
                [zs[q][:, c * HALF:(c + 1) * HALF] for q in range(PK)], axis=-1
            )
        dinv_ref[...] = dinv

    return pl.pallas_call(
        body,
        grid=(GRID,),
        in_specs=[
            pl.BlockSpec((NC, BROW, 128), lambda i: (0, i, 0)),
            pl.BlockSpec((PK, BROW, N_X), lambda i: (0, i, 0)),
            pl.BlockSpec((N_X, DIM_H), lambda i: (0, 0)),
        ],
        out_specs=[
            pl.BlockSpec((NC, BROW, 128), lambda i: (0, i, 0)),
            pl.BlockSpec((BROW, 128), lambda i: (i, 0)),
        ],
        out_shape=[
            jax.ShapeDtypeStruct((NC, NROW, 128), jnp.float32),
            jax.ShapeDtypeStruct((NROW, 128), jnp.float32),
        ],
    )(deg_p, x_perm, W1)


def _tc_mid(p, y, dinv, b_pack, BDW):
    def body(p_ref, y_ref, dinv_ref, b_ref, w_ref, o_ref):
        dinv = dinv_ref[...]
        hs = [
            jnp.maximum(dinv * (p_ref[c] + y_ref[c]) + b_ref[c], 0.0)
            for c in range(NC)
        ]
        for co in range(NC):
            z = sum(
                jnp.dot(hs[c], w_ref[c, co], preferred_element_type=jnp.float32)
                for c in range(NC)
            )
            o_ref[co] = z * dinv

    return pl.pallas_call(
        body,
        grid=(GRID,),
        in_specs=[
            pl.BlockSpec((NC, BROW, 128), lambda i: (0, i, 0)),
            pl.BlockSpec((NC, BROW, 128), lambda i: (0, i, 0)),
            pl.BlockSpec((BROW, 128), lambda i: (i, 0)),
            pl.BlockSpec((NC, 1, 128), lambda i: (0, 0, 0)),
            pl.BlockSpec((NC, NC, 128, 128), lambda i: (0, 0, 0, 0)),
        ],
        out_specs=pl.BlockSpec((NC, BROW, 128), lambda i: (0, i, 0)),
        out_shape=jax.ShapeDtypeStruct((NC, NROW, 128), jnp.float32),
    )(p, y, dinv, b_pack, BDW)


def _tc_tail(p, y, dinv, b_pack, BDWl, bl_pack):
    def body(p_ref, y_ref, dinv_ref, b_ref, wl_ref, bl_ref, o_ref):
        dinv = dinv_ref[...]
        z = bl_ref[...]
        for c in range(NC):
            h = jnp.maximum(dinv * (p_ref[c] + y_ref[c]) + b_ref[c], 0.0)
            z = z + jnp.dot(h, wl_ref[c], preferred_element_type=jnp.float32)
        o_ref[...] = z

    return pl.pallas_call(
        body,
        grid=(GRID,),
        in_specs=[
            pl.BlockSpec((NC, BROW, 128), lambda i: (0, i, 0)),
            pl.BlockSpec((NC, BROW, 128), lambda i: (0, i, 0)),
            pl.BlockSpec((BROW, 128), lambda i: (i, 0)),
            pl.BlockSpec((NC, 1, 128), lambda i: (0, 0, 0)),
            pl.BlockSpec((NC, 128, PK * N_Y), lambda i: (0, 0, 0)),
            pl.BlockSpec((1, PK * N_Y), lambda i: (0, 0)),
        ],
        out_specs=pl.BlockSpec((BROW, PK * N_Y), lambda i: (i, 0)),
        out_shape=jax.ShapeDtypeStruct((NROW, PK * N_Y), jnp.float32),
    )(p, y, dinv, b_pack, BDWl, bl_pack)


def _pack_b(b):
    return jnp.stack([jnp.tile(b[c * HALF:(c + 1) * HALF], PK)[None] for c in range(NC)])


def kernel(x, edge_index, batch, W1, b1, W2, b2, W3, b3, Wl, bl):
    del batch
    src = edge_index[0].astype(jnp.int32)
    dst = edge_index[1].astype(jnp.int32)
    nfull = NS * FCHUNKS * CH
    src_r = (src[:nfull].reshape(NS, FCHUNKS, CH), src[nfull:].reshape(TAIL // CH, CH))
    dst_r = (dst[:nfull].reshape(NS, FCHUNKS, CH), dst[nfull:].reshape(TAIL // CH, CH))
    dst_d = dst.reshape(NC, NS, DCHUNKS, DCH)
    x_perm = (
        jnp.pad(x, ((0, N_PAD - N), (0, 0))).reshape(NROW, PK, N_X).transpose(1, 0, 2)
    )
    eye = jnp.eye(PK, dtype=jnp.float32)
    BDW2 = jnp.stack([
        jnp.stack([jnp.kron(eye, W2[c * HALF:(c + 1) * HALF, co * HALF:(co + 1) * HALF])
                   for co in range(NC)]) for c in range(NC)])
    BDW3 = jnp.stack([
        jnp.stack([jnp.kron(eye, W3[c * HALF:(c + 1) * HALF, co * HALF:(co + 1) * HALF])
                   for co in range(NC)]) for c in range(NC)])
    BDWl = jnp.stack([jnp.kron(eye, Wl[c * HALF:(c + 1) * HALF]) for c in range(NC)])
    bl_pack = jnp.tile(bl, PK)[None]

    deg_p = _sc_deg(dst_d).reshape(NC, NROW, 128)
    y1, dinv = _tc_head(deg_p, x_perm, W1)
    p1 = _sc_agg(y1.reshape(NC, N_PAD, HALF), src_r, dst_r).reshape(NC, NROW, 128)
    y2 = _tc_mid(p1, y1, dinv, _pack_b(b1), BDW2)
    p2 = _sc_agg(y2.reshape(NC, N_PAD, HALF), src_r, dst_r).reshape(NC, NROW, 128)
    y3 = _tc_mid(p2, y2, dinv, _pack_b(b2), BDW3)
    p3 = _sc_agg(y3.reshape(NC, N_PAD, HALF), src_r, dst_r).reshape(NC, NROW, 128)
    out = _tc_tail(p3, y3, dinv, _pack_b(b3), BDWl, bl_pack)
    return out.reshape(N_PAD, N_Y)[:N]
